# baseline (speedup 1.0000x reference)
import functools

import jax
import jax.numpy as jnp
from jax import lax
from jax.experimental import pallas as pl
from jax.experimental.pallas import tpu as pltpu
from jax.experimental.pallas import tpu_sc as plsc

N_PROT = 4096
N_LIG = 1024
N_ALL = N_PROT + N_LIG
E_LIG = 2048
N_GRAPHS = 4
NODE_DIM = 128
EDGE_DIM = 64
TIME_DIM = 16
CLASS_DIM = 8
CLASS_EMB_DIM = 16
NUM_TIMESTEPS = 1000
K_NN = 32

LIG_BLK = 128
N_LIG_BLKS = N_LIG // LIG_BLK
EDGE_BLK = 512
N_EDGE_BLKS = E_LIG // EDGE_BLK
TBL_W = 144

SC_NC = 2
SC_NS = 16
SC_NW = SC_NC * SC_NS
GATHER_B = N_LIG * K_NN
GATHER_CH = 128
GATHER_NCH = GATHER_B // (SC_NW * GATHER_CH)


def _silu(x):
    return x * jax.nn.sigmoid(x)


def _embed_body(prot_ref, wpc_ref, lig_ref, wl_ref, lb_ref, g4_ref, e127_ref,
                a_ref, b_ref, be1_ref, lep_ref, wle_ref, eb_ref, ge4_ref,
                hd_ref, hlig_ref, hsa_ref, eproj_ref):
    oh_lb = (lax.broadcasted_iota(jnp.int32, (N_LIG, N_GRAPHS), 1)
             == lb_ref[...]).astype(jnp.float32)
    hl = (jnp.dot(lig_ref[...], wl_ref[...], preferred_element_type=jnp.float32)
          + jnp.dot(oh_lb, g4_ref[...], preferred_element_type=jnp.float32)
          + e127_ref[...])
    hlig_ref[...] = hl
    hsa_ref[...] = jnp.dot(hl, a_ref[...],
                           preferred_element_type=jnp.float32) + be1_ref[...]
    hd_ref[0:N_PROT, :] = jnp.dot(prot_ref[...], wpc_ref[...],
                                  preferred_element_type=jnp.float32)
    hd_ref[N_PROT:N_ALL, :] = jnp.dot(hl, b_ref[...],
                                      preferred_element_type=jnp.float32)
    oh_eb = (lax.broadcasted_iota(jnp.int32, (E_LIG, N_GRAPHS), 1)
             == eb_ref[...]).astype(jnp.float32)
    eproj_ref[...] = (
        jnp.dot(lep_ref[...], wle_ref[...], preferred_element_type=jnp.float32)
        + jnp.dot(oh_eb, ge4_ref[...], preferred_element_type=jnp.float32))


def _knn_body(ligpos_ref, lb_ref, posT_ref, ab_ref, nbr_ref):
    i = pl.program_id(0)
    xl = ligpos_ref[...]
    posT = posT_ref[...]
    sq_all = jnp.sum(posT * posT, axis=0, keepdims=True)
    sq_lig = jnp.sum(xl * xl, axis=1, keepdims=True)
    d2 = sq_lig + sq_all - 2.0 * jnp.dot(
        xl, posT, preferred_element_type=jnp.float32)
    col = lax.broadcasted_iota(jnp.int32, (LIG_BLK, N_ALL), 1)
    same_batch = ab_ref[...] == lb_ref[...]
    row_gid = N_PROT + i * LIG_BLK + lax.broadcasted_iota(
        jnp.int32, (LIG_BLK, N_ALL), 0)
    inf = jnp.float32(jnp.inf)
    d2 = jnp.where(jnp.logical_and(same_batch, col != row_gid), d2, inf)

    kcol = lax.broadcasted_iota(jnp.int32, (LIG_BLK, K_NN), 1)

    def step(k, carry):
        d2c, acc = carry
        mn = jnp.min(d2c, axis=1, keepdims=True)
        idx = jnp.min(jnp.where(d2c == mn, col, N_ALL), axis=1, keepdims=True)
        acc = jnp.where(kcol == k, idx, acc)
        return jnp.where(col == idx, inf, d2c), acc

    _, nbr = lax.fori_loop(
        0, K_NN, step,
        (d2, jnp.zeros((LIG_BLK, K_NN), jnp.int32)), unroll=False)
    nbr_ref[...] = nbr


def _sc_gather_body(tbl_ref, idx_ref, out_ref, idx_v, rows_v, sem):
    wid = lax.axis_index("s") * SC_NC + lax.axis_index("c")
    base_ch = wid * GATHER_NCH
    pltpu.sync_copy(idx_ref.at[pl.ds(base_ch, GATHER_NCH)], idx_v)
    for c in range(GATHER_NCH):
        pltpu.async_copy(tbl_ref.at[idx_v.at[c]], rows_v, sem).wait()
        pltpu.sync_copy(
            rows_v, out_ref.at[pl.ds((base_ch + c) * GATHER_CH, GATHER_CH)])


def _sc_gather(tbl, idx2):
    run = pl.kernel(
        _sc_gather_body,
        mesh=plsc.VectorSubcoreMesh(core_axis_name="c", subcore_axis_name="s"),
        out_type=jax.ShapeDtypeStruct((GATHER_B, TBL_W), jnp.float32),
        compiler_params=pltpu.CompilerParams(use_tc_tiling_on_sc=False),
        scratch_types=[
            pltpu.VMEM((GATHER_NCH, GATHER_CH), jnp.int32),
            pltpu.VMEM((GATHER_CH, TBL_W), jnp.float32),
            pltpu.SemaphoreType.DMA,
        ],
    )
    return run(tbl, idx2)


def _knn_msg_body(g_ref, hsa_ref, xl_ref, nbr_ref, we2_ref, be2_ref,
                  wx1_ref, bx1_ref, wx2_ref, bx2_ref, v_ref, ck_ref,
                  agg_ref, xagg_ref):
    g = g_ref[...]
    d = g[:, 0:NODE_DIM].reshape(LIG_BLK, K_NN, NODE_DIM)
    pnb = g[:, NODE_DIM:NODE_DIM + 3].reshape(LIG_BLK, K_NN, 3)
    xl = xl_ref[...]
    rel = xl[:, None, :] - pnb
    d2 = jnp.sum(rel * rel, axis=2, keepdims=True)
    m1 = _silu(hsa_ref[...][:, None, :] + d
               + d2 * v_ref[...][None] + ck_ref[...][None])
    m1f = m1.reshape(LIG_BLK * K_NN, NODE_DIM)
    m = _silu(jnp.dot(m1f, we2_ref[...],
                      preferred_element_type=jnp.float32) + be2_ref[...])
    u = _silu(jnp.dot(m, wx1_ref[...],
                      preferred_element_type=jnp.float32) + bx1_ref[...])
    w = jnp.tanh(jnp.sum(u * wx2_ref[...], axis=1, keepdims=True)
                 + bx2_ref[...])
    mask = (nbr_ref[...] < N_PROT).astype(jnp.float32)
    m3 = m.reshape(LIG_BLK, K_NN, NODE_DIM) * mask[:, :, None]
    agg_ref[...] = jnp.sum(m3, axis=1)
    wm = (w.reshape(LIG_BLK, K_NN, 1) * mask[:, :, None])
    xagg_ref[...] = jnp.sum(rel * wm, axis=1)


def _bond_body(srcc_ref, srcr_ref, dstc_ref, dstr_ref, hsa_ref, hdl_ref,
                    xlig_ref, eproj_ref, we2_ref, be2_ref, wx1_ref, bx1_ref,
                    wx2_ref, bx2_ref, v_ref, wed_ref, bed_ref, wq1_ref,
                    bq1_ref, wq2_ref, bq2_ref, pe_ref, aggb_ref, xaggb_ref):
    i = pl.program_id(0)
    oh_s = (lax.broadcasted_iota(jnp.int32, (EDGE_BLK, N_LIG), 1)
            == srcc_ref[...]).astype(jnp.float32)
    oh_d = (lax.broadcasted_iota(jnp.int32, (EDGE_BLK, N_LIG), 1)
            == dstc_ref[...]).astype(jnp.float32)
    hs = jnp.dot(oh_s, hsa_ref[...], preferred_element_type=jnp.float32)
    hd = jnp.dot(oh_d, hdl_ref[...], preferred_element_type=jnp.float32)
    xs = jnp.dot(oh_s, xlig_ref[...], preferred_element_type=jnp.float32)
    xd = jnp.dot(oh_d, xlig_ref[...], preferred_element_type=jnp.float32)
    rel = xs - xd
    d2 = jnp.sum(rel * rel, axis=1, keepdims=True)
    m1 = _silu(hs + hd + d2 * v_ref[...] + eproj_ref[...])
    m = _silu(jnp.dot(m1, we2_ref[...],
                      preferred_element_type=jnp.float32) + be2_ref[...])
    u = _silu(jnp.dot(m, wx1_ref[...],
                      preferred_element_type=jnp.float32) + bx1_ref[...])
    w = jnp.tanh(jnp.sum(u * wx2_ref[...], axis=1, keepdims=True)
                 + bx2_ref[...])
    oh_sT = (lax.broadcasted_iota(jnp.int32, (N_LIG, EDGE_BLK), 0)
             == srcr_ref[...]).astype(jnp.float32)

    @pl.when(i == 0)
    def _():
        aggb_ref[...] = jnp.zeros_like(aggb_ref)
        xaggb_ref[...] = jnp.zeros_like(xaggb_ref)

    aggb_ref[...] += jnp.dot(oh_sT, m, preferred_element_type=jnp.float32)
    xaggb_ref[...] += jnp.dot(oh_sT, rel * w,
                              preferred_element_type=jnp.float32)
    en = _silu(jnp.dot(m, wed_ref[...],
                       preferred_element_type=jnp.float32) + bed_ref[...])
    pe1 = jax.nn.relu(jnp.dot(en, wq1_ref[...],
                              preferred_element_type=jnp.float32) + bq1_ref[...])
    pe_ref[...] = jnp.dot(pe1, wq2_ref[...],
                          preferred_element_type=jnp.float32) + bq2_ref[...]


def _final_body(hlig_ref, aggk_ref, aggb_ref, xaggk_ref, xaggb_ref, xlig_ref,
                wh1a_ref, wh1b_ref, bh1_ref, wh2_ref, bh2_ref,
                wd1_ref, bd1_ref, wd2_ref, bd2_ref, pn_ref, pp_ref):
    hl = hlig_ref[...]
    agg = aggk_ref[...] + aggb_ref[...]
    t1 = _silu(jnp.dot(hl, wh1a_ref[...], preferred_element_type=jnp.float32)
               + jnp.dot(agg, wh1b_ref[...],
                         preferred_element_type=jnp.float32) + bh1_ref[...])
    hn = hl + jnp.dot(t1, wh2_ref[...],
                      preferred_element_type=jnp.float32) + bh2_ref[...]
    pn1 = jax.nn.relu(jnp.dot(hn, wd1_ref[...],
                              preferred_element_type=jnp.float32) + bd1_ref[...])
    pn_ref[...] = jnp.dot(pn1, wd2_ref[...],
                          preferred_element_type=jnp.float32) + bd2_ref[...]
    pp_ref[...] = xlig_ref[...] + (xaggk_ref[...] + xaggb_ref[...]) / K_NN


def kernel(protein_node, protein_pos, protein_batch, ligand_node_pert,
           ligand_pos_pert, ligand_batch, ligand_edge_pert, ligand_edge_index,
           ligand_edge_batch, t, lab, params):
    p = params
    f32 = jnp.float32

    offset = jnp.linspace(0.0, float(NUM_TIMESTEPS), TIME_DIM)
    coeff = -0.5 / (offset[1] - offset[0]) ** 2
    te4 = jnp.exp(coeff * (t[:, None].astype(f32) - offset[None, :]) ** 2)
    ch = lab @ p['Wc1'] + p['bc1']
    mu = ch.mean(-1, keepdims=True)
    var = ch.var(-1, keepdims=True)
    ch = (ch - mu) / jnp.sqrt(var + 1e-5) * p['ln_g'] + p['ln_b']
    ce4 = jax.nn.gelu(ch) @ p['Wc2'] + p['bc2']

    we1 = p['We1']
    A = we1[0:NODE_DIM]
    B = we1[NODE_DIM:2 * NODE_DIM]
    vrow = we1[2 * NODE_DIM:2 * NODE_DIM + 1]
    R = we1[2 * NODE_DIM + 1:]

    nd_l = NODE_DIM - 1 - TIME_DIM - CLASS_EMB_DIM
    g4 = jnp.concatenate(
        [jnp.zeros((N_GRAPHS, nd_l), f32), te4, ce4,
         jnp.zeros((N_GRAPHS, 1), f32)], axis=1)
    wl_pad = jnp.concatenate(
        [p['W_lig_node'], jnp.zeros((p['W_lig_node'].shape[0],
                                     NODE_DIM - nd_l), f32)], axis=1)
    e127 = jnp.zeros((1, NODE_DIM), f32).at[0, NODE_DIM - 1].set(1.0)
    wprot_pad = jnp.concatenate(
        [p['W_prot_node'], jnp.zeros((p['W_prot_node'].shape[0], 1), f32)],
        axis=1)
    wprot_comb = wprot_pad @ B
    ed_l = EDGE_DIM - TIME_DIM - CLASS_EMB_DIM
    wle_comb = p['W_lig_edge'] @ R[0:ed_l]
    ge4 = jnp.concatenate([te4, ce4], axis=1) @ R[ed_l:]
    cknn = (p['W_prot_edge'][0] @ R)[None, :]
    be1 = p['be1'][None, :]

    all_pos = jnp.concatenate([protein_pos, ligand_pos_pert], 0)
    all_batch = jnp.concatenate([protein_batch, ligand_batch], 0)
    lb_col = ligand_batch[:, None].astype(jnp.int32)
    eb_col = ligand_edge_batch[:, None].astype(jnp.int32)

    hd, hlig, hsa, eproj = pl.pallas_call(
        _embed_body,
        out_shape=[
            jax.ShapeDtypeStruct((N_ALL, NODE_DIM), f32),
            jax.ShapeDtypeStruct((N_LIG, NODE_DIM), f32),
            jax.ShapeDtypeStruct((N_LIG, NODE_DIM), f32),
            jax.ShapeDtypeStruct((E_LIG, NODE_DIM), f32),
        ],
    )(protein_node, wprot_comb, ligand_node_pert, wl_pad, lb_col, g4, e127,
      A, B, be1, ligand_edge_pert, wle_comb, eb_col, ge4)

    nbr = pl.pallas_call(
        _knn_body,
        grid=(N_LIG_BLKS,),
        in_specs=[
            pl.BlockSpec((LIG_BLK, 3), lambda i: (i, 0)),
            pl.BlockSpec((LIG_BLK, 1), lambda i: (i, 0)),
            pl.BlockSpec((3, N_ALL), lambda i: (0, 0)),
            pl.BlockSpec((1, N_ALL), lambda i: (0, 0)),
        ],
        out_specs=pl.BlockSpec((LIG_BLK, K_NN), lambda i: (i, 0)),
        out_shape=jax.ShapeDtypeStruct((N_LIG, K_NN), jnp.int32),
    )(ligand_pos_pert, lb_col, all_pos.T, all_batch[None, :].astype(jnp.int32))

    tbl = jnp.concatenate(
        [hd, all_pos, jnp.zeros((N_ALL, TBL_W - NODE_DIM - 3), f32)], axis=1)
    idx2 = nbr.reshape(GATHER_B // GATHER_CH, GATHER_CH)
    g = _sc_gather(tbl, idx2)

    wx2row = p['Wx2'][:, 0][None, :]
    bx2 = p['bx2'][None, :]
    aggk, xaggk = pl.pallas_call(
        _knn_msg_body,
        grid=(N_LIG_BLKS,),
        in_specs=[
            pl.BlockSpec((LIG_BLK * K_NN, TBL_W), lambda i: (i, 0)),
            pl.BlockSpec((LIG_BLK, NODE_DIM), lambda i: (i, 0)),
            pl.BlockSpec((LIG_BLK, 3), lambda i: (i, 0)),
            pl.BlockSpec((LIG_BLK, K_NN), lambda i: (i, 0)),
            pl.BlockSpec((NODE_DIM, NODE_DIM), lambda i: (0, 0)),
            pl.BlockSpec((1, NODE_DIM), lambda i: (0, 0)),
            pl.BlockSpec((NODE_DIM, NODE_DIM), lambda i: (0, 0)),
            pl.BlockSpec((1, NODE_DIM), lambda i: (0, 0)),
            pl.BlockSpec((1, NODE_DIM), lambda i: (0, 0)),
            pl.BlockSpec((1, 1), lambda i: (0, 0)),
            pl.BlockSpec((1, NODE_DIM), lambda i: (0, 0)),
            pl.BlockSpec((1, NODE_DIM), lambda i: (0, 0)),
        ],
        out_specs=[
            pl.BlockSpec((LIG_BLK, NODE_DIM), lambda i: (i, 0)),
            pl.BlockSpec((LIG_BLK, 3), lambda i: (i, 0)),
        ],
        out_shape=[
            jax.ShapeDtypeStruct((N_LIG, NODE_DIM), f32),
            jax.ShapeDtypeStruct((N_LIG, 3), f32),
        ],
    )(g, hsa, ligand_pos_pert, nbr, p['We2'], p['be2'][None, :],
      p['Wx1'], p['bx1'][None, :], wx2row, bx2, vrow, cknn)

    src_l = ligand_edge_index[0].astype(jnp.int32)
    dst_l = ligand_edge_index[1].astype(jnp.int32)
    hdl = hd[N_PROT:]
    wspec = pl.BlockSpec((NODE_DIM, NODE_DIM), lambda i: (0, 0))
    rspec = pl.BlockSpec((1, NODE_DIM), lambda i: (0, 0))
    pe, aggb, xaggb = pl.pallas_call(
        _bond_body,
        grid=(N_EDGE_BLKS,),
        in_specs=[
            pl.BlockSpec((EDGE_BLK, 1), lambda i: (i, 0)),
            pl.BlockSpec((1, EDGE_BLK), lambda i: (0, i)),
            pl.BlockSpec((EDGE_BLK, 1), lambda i: (i, 0)),
            pl.BlockSpec((1, EDGE_BLK), lambda i: (0, i)),
            pl.BlockSpec((N_LIG, NODE_DIM), lambda i: (0, 0)),
            pl.BlockSpec((N_LIG, NODE_DIM), lambda i: (0, 0)),
            pl.BlockSpec((N_LIG, 3), lambda i: (0, 0)),
            pl.BlockSpec((EDGE_BLK, NODE_DIM), lambda i: (i, 0)),
            wspec, rspec, wspec, rspec,
            rspec, pl.BlockSpec((1, 1), lambda i: (0, 0)), rspec,
            pl.BlockSpec((NODE_DIM, EDGE_DIM), lambda i: (0, 0)),
            pl.BlockSpec((1, EDGE_DIM), lambda i: (0, 0)),
            pl.BlockSpec((EDGE_DIM, EDGE_DIM), lambda i: (0, 0)),
            pl.BlockSpec((1, EDGE_DIM), lambda i: (0, 0)),
            pl.BlockSpec((EDGE_DIM, 5), lambda i: (0, 0)),
            pl.BlockSpec((1, 5), lambda i: (0, 0)),
        ],
        out_specs=[
            pl.BlockSpec((EDGE_BLK, 5), lambda i: (i, 0)),
            pl.BlockSpec((N_LIG, NODE_DIM), lambda i: (0, 0)),
            pl.BlockSpec((N_LIG, 3), lambda i: (0, 0)),
        ],
        out_shape=[
            jax.ShapeDtypeStruct((E_LIG, 5), f32),
            jax.ShapeDtypeStruct((N_LIG, NODE_DIM), f32),
            jax.ShapeDtypeStruct((N_LIG, 3), f32),
        ],
    )(src_l[:, None], src_l[None, :], dst_l[:, None], dst_l[None, :],
      hsa, hdl, ligand_pos_pert, eproj,
      p['We2'], p['be2'][None, :], p['Wx1'], p['bx1'][None, :],
      wx2row, bx2, vrow,
      p['Wed'], p['bed'][None, :], p['Wq1'], p['bq1'][None, :],
      p['Wq2'], p['bq2'][None, :])

    pn, pp = pl.pallas_call(
        _final_body,
        out_shape=[
            jax.ShapeDtypeStruct((N_LIG, p['Wd2'].shape[1]), f32),
            jax.ShapeDtypeStruct((N_LIG, 3), f32),
        ],
    )(hlig, aggk, aggb, xaggk, xaggb, ligand_pos_pert,
      p['Wh1'][0:NODE_DIM], p['Wh1'][NODE_DIM:], p['bh1'][None, :],
      p['Wh2'], p['bh2'][None, :], p['Wd1'], p['bd1'][None, :],
      p['Wd2'], p['bd2'][None, :])

    return pn, pp, pe


# speedup vs baseline: 1.0840x; 1.0840x over previous
import functools

import jax
import jax.numpy as jnp
from jax import lax
from jax.experimental import pallas as pl
from jax.experimental.pallas import tpu as pltpu
from jax.experimental.pallas import tpu_sc as plsc

N_PROT = 4096
N_LIG = 1024
N_ALL = N_PROT + N_LIG
E_LIG = 2048
N_GRAPHS = 4
NODE_DIM = 128
EDGE_DIM = 64
TIME_DIM = 16
CLASS_DIM = 8
CLASS_EMB_DIM = 16
NUM_TIMESTEPS = 1000
K_NN = 32

LIG_BLK = 128
N_LIG_BLKS = N_LIG // LIG_BLK
EDGE_BLK = 512
N_EDGE_BLKS = E_LIG // EDGE_BLK
TBL_W = 256

SC_NC = 2
SC_NS = 16
SC_NW = SC_NC * SC_NS
GATHER_B = N_LIG * K_NN
GATHER_CH = 128
GATHER_NCH = GATHER_B // (SC_NW * GATHER_CH)


def _silu(x):
    return x * jax.nn.sigmoid(x)


def _embed_body(prot_ref, wpc_ref, lig_ref, wl_ref, lb_ref, g4_ref, e127_ref,
                a_ref, b_ref, be1_ref, lep_ref, wle_ref, eb_ref, ge4_ref,
                pos16_ref, hd_ref, hlig_ref, hsa_ref, eproj_ref):
    oh_lb = (lax.broadcasted_iota(jnp.int32, (N_LIG, N_GRAPHS), 1)
             == lb_ref[...]).astype(jnp.float32)
    hl = (jnp.dot(lig_ref[...], wl_ref[...], preferred_element_type=jnp.float32)
          + jnp.dot(oh_lb, g4_ref[...], preferred_element_type=jnp.float32)
          + e127_ref[...])
    hlig_ref[...] = hl
    hsa_ref[...] = jnp.dot(hl, a_ref[...],
                           preferred_element_type=jnp.float32) + be1_ref[...]
    hd_ref[0:N_PROT, 0:NODE_DIM] = jnp.dot(
        prot_ref[...], wpc_ref[...], preferred_element_type=jnp.float32)
    hd_ref[N_PROT:N_ALL, 0:NODE_DIM] = jnp.dot(
        hl, b_ref[...], preferred_element_type=jnp.float32)
    hd_ref[:, NODE_DIM:TBL_W] = pos16_ref[...]
    oh_eb = (lax.broadcasted_iota(jnp.int32, (E_LIG, N_GRAPHS), 1)
             == eb_ref[...]).astype(jnp.float32)
    eproj_ref[...] = (
        jnp.dot(lep_ref[...], wle_ref[...], preferred_element_type=jnp.float32)
        + jnp.dot(oh_eb, ge4_ref[...], preferred_element_type=jnp.float32))


def _knn_body(ligpos_ref, lb_ref, posT_ref, ab_ref, nbr_ref):
    i = pl.program_id(0)
    xl = ligpos_ref[...]
    posT = posT_ref[...]
    sq_all = jnp.sum(posT * posT, axis=0, keepdims=True)
    sq_lig = jnp.sum(xl * xl, axis=1, keepdims=True)
    d2 = sq_lig + sq_all - 2.0 * jnp.dot(
        xl, posT, preferred_element_type=jnp.float32)
    col = lax.broadcasted_iota(jnp.int32, (LIG_BLK, N_ALL), 1)
    same_batch = ab_ref[...] == lb_ref[...]
    row_gid = N_PROT + i * LIG_BLK + lax.broadcasted_iota(
        jnp.int32, (LIG_BLK, N_ALL), 0)
    inf = jnp.float32(jnp.inf)
    d2 = jnp.where(jnp.logical_and(same_batch, col != row_gid), d2, inf)
    kcol = lax.broadcasted_iota(jnp.int32, (LIG_BLK, K_NN), 1)

    def step(k, carry):
        d2c, acc = carry
        mn = jnp.min(d2c, axis=1, keepdims=True)
        idx = jnp.min(jnp.where(d2c == mn, col, N_ALL), axis=1, keepdims=True)
        acc = jnp.where(kcol == k, idx, acc)
        return jnp.where(col == idx, inf, d2c), acc

    _, nbr = lax.fori_loop(
        0, K_NN, step,
        (d2, jnp.zeros((LIG_BLK, K_NN), jnp.int32)), unroll=False)
    nbr_ref[...] = nbr


def _sc_gather_body(tbl_ref, idx_ref, out_ref, idx_v, rows_v, sem):
    wid = lax.axis_index("s") * SC_NC + lax.axis_index("c")
    base_ch = wid * GATHER_NCH
    pltpu.sync_copy(idx_ref.at[pl.ds(base_ch, GATHER_NCH)], idx_v)
    for c in range(GATHER_NCH):
        pltpu.async_copy(tbl_ref.at[idx_v.at[c]], rows_v, sem).wait()
        pltpu.sync_copy(
            rows_v, out_ref.at[pl.ds((base_ch + c) * GATHER_CH, GATHER_CH)])


def _sc_gather(tbl, idx2):
    run = pl.kernel(
        _sc_gather_body,
        mesh=plsc.VectorSubcoreMesh(core_axis_name="c", subcore_axis_name="s"),
        out_type=jax.ShapeDtypeStruct((GATHER_B, TBL_W), jnp.float32),
        scratch_types=[
            pltpu.VMEM((GATHER_NCH, GATHER_CH), jnp.int32),
            pltpu.VMEM((GATHER_CH, TBL_W), jnp.float32),
            pltpu.SemaphoreType.DMA,
        ],
    )
    return run(tbl, idx2)


def _knn_msg_body(g_ref, hsa_ref, xl_ref, nbr_ref, we2_ref, be2_ref,
                  wx1_ref, bx1_ref, wx2_ref, bx2_ref, v_ref, ck_ref,
                  aggb_ref, xaggb_ref, hlig_ref, wh1a_ref, wh1b_ref, bh1_ref,
                  wh2_ref, bh2_ref, wd1_ref, bd1_ref, wd2_ref, bd2_ref,
                  pn_ref, pp_ref):
    g = g_ref[...]
    d = g[:, 0:NODE_DIM].reshape(LIG_BLK, K_NN, NODE_DIM)
    pnb = g[:, NODE_DIM:NODE_DIM + 3].reshape(LIG_BLK, K_NN, 3)
    xl = xl_ref[...]
    rel = xl[:, None, :] - pnb
    d2 = jnp.sum(rel * rel, axis=2, keepdims=True)
    m1 = _silu(hsa_ref[...][:, None, :] + d
               + d2 * v_ref[...][None] + ck_ref[...][None])
    m1f = m1.reshape(LIG_BLK * K_NN, NODE_DIM)
    m = _silu(jnp.dot(m1f, we2_ref[...],
                      preferred_element_type=jnp.float32) + be2_ref[...])
    u = _silu(jnp.dot(m, wx1_ref[...],
                      preferred_element_type=jnp.float32) + bx1_ref[...])
    w = jnp.tanh(jnp.sum(u * wx2_ref[...], axis=1, keepdims=True)
                 + bx2_ref[...])
    mask = (nbr_ref[...] < N_PROT).astype(jnp.float32)
    m3 = m.reshape(LIG_BLK, K_NN, NODE_DIM) * mask[:, :, None]
    agg = jnp.sum(m3, axis=1) + aggb_ref[...]
    wm = (w.reshape(LIG_BLK, K_NN, 1) * mask[:, :, None])
    xagg = jnp.sum(rel * wm, axis=1) + xaggb_ref[...]
    hl = hlig_ref[...]
    t1 = _silu(jnp.dot(hl, wh1a_ref[...], preferred_element_type=jnp.float32)
               + jnp.dot(agg, wh1b_ref[...],
                         preferred_element_type=jnp.float32) + bh1_ref[...])
    hn = hl + jnp.dot(t1, wh2_ref[...],
                      preferred_element_type=jnp.float32) + bh2_ref[...]
    pn1 = jax.nn.relu(jnp.dot(hn, wd1_ref[...],
                              preferred_element_type=jnp.float32) + bd1_ref[...])
    pn_ref[...] = jnp.dot(pn1, wd2_ref[...],
                          preferred_element_type=jnp.float32) + bd2_ref[...]
    pp_ref[...] = xl + xagg / K_NN


def _bond_body(srcc_ref, srcr_ref, dstc_ref, dstr_ref, hsa_ref, hdl_ref,
                    xlig_ref, eproj_ref, we2_ref, be2_ref, wx1_ref, bx1_ref,
                    wx2_ref, bx2_ref, v_ref, wed_ref, bed_ref, wq1_ref,
                    bq1_ref, wq2_ref, bq2_ref, pe_ref, aggb_ref, xaggb_ref):
    i = pl.program_id(0)
    oh_s = (lax.broadcasted_iota(jnp.int32, (EDGE_BLK, N_LIG), 1)
            == srcc_ref[...]).astype(jnp.float32)
    oh_d = (lax.broadcasted_iota(jnp.int32, (EDGE_BLK, N_LIG), 1)
            == dstc_ref[...]).astype(jnp.float32)
    hs = jnp.dot(oh_s, hsa_ref[...], preferred_element_type=jnp.float32)
    hd = jnp.dot(oh_d, hdl_ref[...], preferred_element_type=jnp.float32)
    xs = jnp.dot(oh_s, xlig_ref[...], preferred_element_type=jnp.float32)
    xd = jnp.dot(oh_d, xlig_ref[...], preferred_element_type=jnp.float32)
    rel = xs - xd
    d2 = jnp.sum(rel * rel, axis=1, keepdims=True)
    m1 = _silu(hs + hd + d2 * v_ref[...] + eproj_ref[...])
    m = _silu(jnp.dot(m1, we2_ref[...],
                      preferred_element_type=jnp.float32) + be2_ref[...])
    u = _silu(jnp.dot(m, wx1_ref[...],
                      preferred_element_type=jnp.float32) + bx1_ref[...])
    w = jnp.tanh(jnp.sum(u * wx2_ref[...], axis=1, keepdims=True)
                 + bx2_ref[...])
    oh_sT = (lax.broadcasted_iota(jnp.int32, (N_LIG, EDGE_BLK), 0)
             == srcr_ref[...]).astype(jnp.float32)

    @pl.when(i == 0)
    def _():
        aggb_ref[...] = jnp.zeros_like(aggb_ref)
        xaggb_ref[...] = jnp.zeros_like(xaggb_ref)

    aggb_ref[...] += jnp.dot(oh_sT, m, preferred_element_type=jnp.float32)
    xaggb_ref[...] += jnp.dot(oh_sT, rel * w,
                              preferred_element_type=jnp.float32)
    en = _silu(jnp.dot(m, wed_ref[...],
                       preferred_element_type=jnp.float32) + bed_ref[...])
    pe1 = jax.nn.relu(jnp.dot(en, wq1_ref[...],
                              preferred_element_type=jnp.float32) + bq1_ref[...])
    pe_ref[...] = jnp.dot(pe1, wq2_ref[...],
                          preferred_element_type=jnp.float32) + bq2_ref[...]


def kernel(protein_node, protein_pos, protein_batch, ligand_node_pert,
           ligand_pos_pert, ligand_batch, ligand_edge_pert, ligand_edge_index,
           ligand_edge_batch, t, lab, params):
    p = params
    f32 = jnp.float32

    offset = jnp.linspace(0.0, float(NUM_TIMESTEPS), TIME_DIM)
    coeff = -0.5 / (offset[1] - offset[0]) ** 2
    te4 = jnp.exp(coeff * (t[:, None].astype(f32) - offset[None, :]) ** 2)
    ch = lab @ p['Wc1'] + p['bc1']
    mu = ch.mean(-1, keepdims=True)
    var = ch.var(-1, keepdims=True)
    ch = (ch - mu) / jnp.sqrt(var + 1e-5) * p['ln_g'] + p['ln_b']
    ce4 = jax.nn.gelu(ch) @ p['Wc2'] + p['bc2']

    we1 = p['We1']
    A = we1[0:NODE_DIM]
    B = we1[NODE_DIM:2 * NODE_DIM]
    vrow = we1[2 * NODE_DIM:2 * NODE_DIM + 1]
    R = we1[2 * NODE_DIM + 1:]

    nd_l = NODE_DIM - 1 - TIME_DIM - CLASS_EMB_DIM
    g4 = jnp.concatenate(
        [jnp.zeros((N_GRAPHS, nd_l), f32), te4, ce4,
         jnp.zeros((N_GRAPHS, 1), f32)], axis=1)
    wl_pad = jnp.concatenate(
        [p['W_lig_node'], jnp.zeros((p['W_lig_node'].shape[0],
                                     NODE_DIM - nd_l), f32)], axis=1)
    e127 = jnp.zeros((1, NODE_DIM), f32).at[0, NODE_DIM - 1].set(1.0)
    wprot_pad = jnp.concatenate(
        [p['W_prot_node'], jnp.zeros((p['W_prot_node'].shape[0], 1), f32)],
        axis=1)
    wprot_comb = wprot_pad @ B
    ed_l = EDGE_DIM - TIME_DIM - CLASS_EMB_DIM
    wle_comb = p['W_lig_edge'] @ R[0:ed_l]
    ge4 = jnp.concatenate([te4, ce4], axis=1) @ R[ed_l:]
    cknn = (p['W_prot_edge'][0] @ R)[None, :]
    be1 = p['be1'][None, :]

    all_pos = jnp.concatenate([protein_pos, ligand_pos_pert], 0)
    all_batch = jnp.concatenate([protein_batch, ligand_batch], 0)
    lb_col = ligand_batch[:, None].astype(jnp.int32)
    eb_col = ligand_edge_batch[:, None].astype(jnp.int32)

    pos16 = jnp.concatenate(
        [all_pos, jnp.zeros((N_ALL, TBL_W - NODE_DIM - 3), f32)], axis=1)
    tbl, hlig, hsa, eproj = pl.pallas_call(
        _embed_body,
        out_shape=[
            jax.ShapeDtypeStruct((N_ALL, TBL_W), f32),
            jax.ShapeDtypeStruct((N_LIG, NODE_DIM), f32),
            jax.ShapeDtypeStruct((N_LIG, NODE_DIM), f32),
            jax.ShapeDtypeStruct((E_LIG, NODE_DIM), f32),
        ],
    )(protein_node, wprot_comb, ligand_node_pert, wl_pad, lb_col, g4, e127,
      A, B, be1, ligand_edge_pert, wle_comb, eb_col, ge4, pos16)

    nbr = pl.pallas_call(
        _knn_body,
        grid=(N_LIG_BLKS,),
        in_specs=[
            pl.BlockSpec((LIG_BLK, 3), lambda i: (i, 0)),
            pl.BlockSpec((LIG_BLK, 1), lambda i: (i, 0)),
            pl.BlockSpec((3, N_ALL), lambda i: (0, 0)),
            pl.BlockSpec((1, N_ALL), lambda i: (0, 0)),
        ],
        out_specs=pl.BlockSpec((LIG_BLK, K_NN), lambda i: (i, 0)),
        out_shape=jax.ShapeDtypeStruct((N_LIG, K_NN), jnp.int32),
    )(ligand_pos_pert, lb_col, all_pos.T, all_batch[None, :].astype(jnp.int32))

    idx2 = nbr.reshape(GATHER_B // GATHER_CH, GATHER_CH)
    g = _sc_gather(tbl, idx2)

    wx2row = p['Wx2'][:, 0][None, :]
    bx2 = p['bx2'][None, :]

    src_l = ligand_edge_index[0].astype(jnp.int32)
    dst_l = ligand_edge_index[1].astype(jnp.int32)
    hdl = tbl[N_PROT:, 0:NODE_DIM]
    wspec = pl.BlockSpec((NODE_DIM, NODE_DIM), lambda i: (0, 0))
    rspec = pl.BlockSpec((1, NODE_DIM), lambda i: (0, 0))
    pe, aggb, xaggb = pl.pallas_call(
        _bond_body,
        grid=(N_EDGE_BLKS,),
        in_specs=[
            pl.BlockSpec((EDGE_BLK, 1), lambda i: (i, 0)),
            pl.BlockSpec((1, EDGE_BLK), lambda i: (0, i)),
            pl.BlockSpec((EDGE_BLK, 1), lambda i: (i, 0)),
            pl.BlockSpec((1, EDGE_BLK), lambda i: (0, i)),
            pl.BlockSpec((N_LIG, NODE_DIM), lambda i: (0, 0)),
            pl.BlockSpec((N_LIG, NODE_DIM), lambda i: (0, 0)),
            pl.BlockSpec((N_LIG, 3), lambda i: (0, 0)),
            pl.BlockSpec((EDGE_BLK, NODE_DIM), lambda i: (i, 0)),
            wspec, rspec, wspec, rspec,
            rspec, pl.BlockSpec((1, 1), lambda i: (0, 0)), rspec,
            pl.BlockSpec((NODE_DIM, EDGE_DIM), lambda i: (0, 0)),
            pl.BlockSpec((1, EDGE_DIM), lambda i: (0, 0)),
            pl.BlockSpec((EDGE_DIM, EDGE_DIM), lambda i: (0, 0)),
            pl.BlockSpec((1, EDGE_DIM), lambda i: (0, 0)),
            pl.BlockSpec((EDGE_DIM, 5), lambda i: (0, 0)),
            pl.BlockSpec((1, 5), lambda i: (0, 0)),
        ],
        out_specs=[
            pl.BlockSpec((EDGE_BLK, 5), lambda i: (i, 0)),
            pl.BlockSpec((N_LIG, NODE_DIM), lambda i: (0, 0)),
            pl.BlockSpec((N_LIG, 3), lambda i: (0, 0)),
        ],
        out_shape=[
            jax.ShapeDtypeStruct((E_LIG, 5), f32),
            jax.ShapeDtypeStruct((N_LIG, NODE_DIM), f32),
            jax.ShapeDtypeStruct((N_LIG, 3), f32),
        ],
    )(src_l[:, None], src_l[None, :], dst_l[:, None], dst_l[None, :],
      hsa, hdl, ligand_pos_pert, eproj,
      p['We2'], p['be2'][None, :], p['Wx1'], p['bx1'][None, :],
      wx2row, bx2, vrow,
      p['Wed'], p['bed'][None, :], p['Wq1'], p['bq1'][None, :],
      p['Wq2'], p['bq2'][None, :])

    wspec2 = pl.BlockSpec((NODE_DIM, NODE_DIM), lambda i: (0, 0))
    rspec2 = pl.BlockSpec((1, NODE_DIM), lambda i: (0, 0))
    pn, pp = pl.pallas_call(
        _knn_msg_body,
        grid=(N_LIG_BLKS,),
        in_specs=[
            pl.BlockSpec((LIG_BLK * K_NN, TBL_W), lambda i: (i, 0)),
            pl.BlockSpec((LIG_BLK, NODE_DIM), lambda i: (i, 0)),
            pl.BlockSpec((LIG_BLK, 3), lambda i: (i, 0)),
            pl.BlockSpec((LIG_BLK, K_NN), lambda i: (i, 0)),
            wspec2, rspec2, wspec2, rspec2, rspec2,
            pl.BlockSpec((1, 1), lambda i: (0, 0)),
            rspec2, rspec2,
            pl.BlockSpec((LIG_BLK, NODE_DIM), lambda i: (i, 0)),
            pl.BlockSpec((LIG_BLK, 3), lambda i: (i, 0)),
            pl.BlockSpec((LIG_BLK, NODE_DIM), lambda i: (i, 0)),
            wspec2, wspec2, rspec2, wspec2, rspec2,
            wspec2, rspec2,
            pl.BlockSpec((NODE_DIM, p['Wd2'].shape[1]), lambda i: (0, 0)),
            pl.BlockSpec((1, p['Wd2'].shape[1]), lambda i: (0, 0)),
        ],
        out_specs=[
            pl.BlockSpec((LIG_BLK, p['Wd2'].shape[1]), lambda i: (i, 0)),
            pl.BlockSpec((LIG_BLK, 3), lambda i: (i, 0)),
        ],
        out_shape=[
            jax.ShapeDtypeStruct((N_LIG, p['Wd2'].shape[1]), f32),
            jax.ShapeDtypeStruct((N_LIG, 3), f32),
        ],
    )(g, hsa, ligand_pos_pert, nbr, p['We2'], p['be2'][None, :],
      p['Wx1'], p['bx1'][None, :], wx2row, bx2, vrow, cknn,
      aggb, xaggb, hlig,
      p['Wh1'][0:NODE_DIM], p['Wh1'][NODE_DIM:], p['bh1'][None, :],
      p['Wh2'], p['bh2'][None, :], p['Wd1'], p['bd1'][None, :],
      p['Wd2'], p['bd2'][None, :])

    return pn, pp, pe


# speedup vs baseline: 1.1255x; 1.0383x over previous
import functools

import jax
import jax.numpy as jnp
from jax import lax
from jax.experimental import pallas as pl
from jax.experimental.pallas import tpu as pltpu
from jax.experimental.pallas import tpu_sc as plsc

N_PROT = 4096
N_LIG = 1024
N_ALL = N_PROT + N_LIG
E_LIG = 2048
N_GRAPHS = 4
NODE_DIM = 128
EDGE_DIM = 64
TIME_DIM = 16
CLASS_DIM = 8
CLASS_EMB_DIM = 16
NUM_TIMESTEPS = 1000
K_NN = 32

LIG_BLK = 128
N_LIG_BLKS = N_LIG // LIG_BLK
EDGE_BLK = 512
N_EDGE_BLKS = E_LIG // EDGE_BLK
TBL_W = 256

SC_NC = 2
SC_NS = 16
SC_NW = SC_NC * SC_NS
GATHER_B = N_LIG * K_NN
GATHER_CH = 128
GATHER_NCH = GATHER_B // (SC_NW * GATHER_CH)


def _silu(x):
    return x * jax.nn.sigmoid(x)


def _embed_body(prot_ref, wpc_ref, lig_ref, wl_ref, lb_ref, g4_ref, e127_ref,
                a_ref, b_ref, be1_ref, lep_ref, wle_ref, eb_ref, ge4_ref,
                pos16_ref, hd_ref, hlig_ref, hsa_ref, eproj_ref):
    oh_lb = (lax.broadcasted_iota(jnp.int32, (N_LIG, N_GRAPHS), 1)
             == lb_ref[...]).astype(jnp.float32)
    hl = (jnp.dot(lig_ref[...], wl_ref[...], preferred_element_type=jnp.float32)
          + jnp.dot(oh_lb, g4_ref[...], preferred_element_type=jnp.float32)
          + e127_ref[...])
    hlig_ref[...] = hl
    hsa_ref[...] = jnp.dot(hl, a_ref[...],
                           preferred_element_type=jnp.float32) + be1_ref[...]
    hd_ref[0:N_PROT, 0:NODE_DIM] = jnp.dot(
        prot_ref[...], wpc_ref[...], preferred_element_type=jnp.float32)
    hd_ref[N_PROT:N_ALL, 0:NODE_DIM] = jnp.dot(
        hl, b_ref[...], preferred_element_type=jnp.float32)
    hd_ref[:, NODE_DIM:TBL_W] = pos16_ref[...]
    oh_eb = (lax.broadcasted_iota(jnp.int32, (E_LIG, N_GRAPHS), 1)
             == eb_ref[...]).astype(jnp.float32)
    eproj_ref[...] = (
        jnp.dot(lep_ref[...], wle_ref[...], preferred_element_type=jnp.float32)
        + jnp.dot(oh_eb, ge4_ref[...], preferred_element_type=jnp.float32))


def _knn_body(ligpos_ref, lb_ref, posT_ref, ab_ref, nbr_ref):
    xl = ligpos_ref[...]
    posT = posT_ref[...]
    sq_all = jnp.sum(posT * posT, axis=0, keepdims=True)
    sq_lig = jnp.sum(xl * xl, axis=1, keepdims=True)
    d2 = sq_lig + sq_all - 2.0 * jnp.dot(
        xl, posT, preferred_element_type=jnp.float32)
    col = lax.broadcasted_iota(jnp.int32, (N_LIG, N_ALL), 1)
    same_batch = ab_ref[...] == lb_ref[...]
    row_gid = N_PROT + lax.broadcasted_iota(
        jnp.int32, (N_LIG, N_ALL), 0)
    inf = jnp.float32(jnp.inf)
    d2 = jnp.where(jnp.logical_and(same_batch, col != row_gid), d2, inf)
    kcol = lax.broadcasted_iota(jnp.int32, (N_LIG, K_NN), 1)

    def step(k, carry):
        d2c, acc = carry
        mn = jnp.min(d2c, axis=1, keepdims=True)
        idx = jnp.min(jnp.where(d2c == mn, col, N_ALL), axis=1, keepdims=True)
        acc = jnp.where(kcol == k, idx, acc)
        return jnp.where(col == idx, inf, d2c), acc

    _, nbr = lax.fori_loop(
        0, K_NN, step,
        (d2, jnp.zeros((N_LIG, K_NN), jnp.int32)), unroll=False)
    nbr_ref[...] = nbr


def _sc_gather_body(tbl_ref, idx_ref, out_ref, idx_v, rows_v, sem):
    wid = lax.axis_index("s") * SC_NC + lax.axis_index("c")
    base_ch = wid * GATHER_NCH
    pltpu.sync_copy(idx_ref.at[pl.ds(base_ch, GATHER_NCH)], idx_v)
    for c in range(GATHER_NCH):
        pltpu.async_copy(tbl_ref.at[idx_v.at[c]], rows_v, sem).wait()
        pltpu.sync_copy(
            rows_v, out_ref.at[pl.ds((base_ch + c) * GATHER_CH, GATHER_CH)])


def _sc_gather(tbl, idx2):
    run = pl.kernel(
        _sc_gather_body,
        mesh=plsc.VectorSubcoreMesh(core_axis_name="c", subcore_axis_name="s"),
        out_type=jax.ShapeDtypeStruct((GATHER_B, TBL_W), jnp.float32),
        scratch_types=[
            pltpu.VMEM((GATHER_NCH, GATHER_CH), jnp.int32),
            pltpu.VMEM((GATHER_CH, TBL_W), jnp.float32),
            pltpu.SemaphoreType.DMA,
        ],
    )
    return run(tbl, idx2)


def _knn_msg_body(g_ref, hsa_ref, xl_ref, nbr_ref, we2_ref, be2_ref,
                  wx1_ref, bx1_ref, wx2_ref, bx2_ref, v_ref, ck_ref,
                  aggb_ref, xaggb_ref, hlig_ref, wh1a_ref, wh1b_ref, bh1_ref,
                  wh2_ref, bh2_ref, wd1_ref, bd1_ref, wd2_ref, bd2_ref,
                  pn_ref, pp_ref):
    g = g_ref[...]
    d = g[:, 0:NODE_DIM].reshape(LIG_BLK, K_NN, NODE_DIM)
    pnb = g[:, NODE_DIM:NODE_DIM + 3].reshape(LIG_BLK, K_NN, 3)
    xl = xl_ref[...]
    rel = xl[:, None, :] - pnb
    d2 = jnp.sum(rel * rel, axis=2, keepdims=True)
    m1 = _silu(hsa_ref[...][:, None, :] + d
               + d2 * v_ref[...][None] + ck_ref[...][None])
    m1f = m1.reshape(LIG_BLK * K_NN, NODE_DIM)
    m = _silu(jnp.dot(m1f, we2_ref[...],
                      preferred_element_type=jnp.float32) + be2_ref[...])
    u = _silu(jnp.dot(m, wx1_ref[...],
                      preferred_element_type=jnp.float32) + bx1_ref[...])
    w = jnp.tanh(jnp.sum(u * wx2_ref[...], axis=1, keepdims=True)
                 + bx2_ref[...])
    mask = (nbr_ref[...] < N_PROT).astype(jnp.float32)
    m3 = m.reshape(LIG_BLK, K_NN, NODE_DIM) * mask[:, :, None]
    agg = jnp.sum(m3, axis=1) + aggb_ref[...]
    wm = (w.reshape(LIG_BLK, K_NN, 1) * mask[:, :, None])
    xagg = jnp.sum(rel * wm, axis=1) + xaggb_ref[...]
    hl = hlig_ref[...]
    t1 = _silu(jnp.dot(hl, wh1a_ref[...], preferred_element_type=jnp.float32)
               + jnp.dot(agg, wh1b_ref[...],
                         preferred_element_type=jnp.float32) + bh1_ref[...])
    hn = hl + jnp.dot(t1, wh2_ref[...],
                      preferred_element_type=jnp.float32) + bh2_ref[...]
    pn1 = jax.nn.relu(jnp.dot(hn, wd1_ref[...],
                              preferred_element_type=jnp.float32) + bd1_ref[...])
    pn_ref[...] = jnp.dot(pn1, wd2_ref[...],
                          preferred_element_type=jnp.float32) + bd2_ref[...]
    pp_ref[...] = xl + xagg / K_NN


def _bond_body(srcc_ref, srcr_ref, dstc_ref, dstr_ref, hsa_ref, hdl_ref,
                    xlig_ref, eproj_ref, we2_ref, be2_ref, wx1_ref, bx1_ref,
                    wx2_ref, bx2_ref, v_ref, wed_ref, bed_ref, wq1_ref,
                    bq1_ref, wq2_ref, bq2_ref, pe_ref, aggb_ref, xaggb_ref):
    i = pl.program_id(0)
    oh_s = (lax.broadcasted_iota(jnp.int32, (EDGE_BLK, N_LIG), 1)
            == srcc_ref[...]).astype(jnp.float32)
    oh_d = (lax.broadcasted_iota(jnp.int32, (EDGE_BLK, N_LIG), 1)
            == dstc_ref[...]).astype(jnp.float32)
    hs = jnp.dot(oh_s, hsa_ref[...], preferred_element_type=jnp.float32)
    hd = jnp.dot(oh_d, hdl_ref[...], preferred_element_type=jnp.float32)
    xs = jnp.dot(oh_s, xlig_ref[...], preferred_element_type=jnp.float32)
    xd = jnp.dot(oh_d, xlig_ref[...], preferred_element_type=jnp.float32)
    rel = xs - xd
    d2 = jnp.sum(rel * rel, axis=1, keepdims=True)
    m1 = _silu(hs + hd + d2 * v_ref[...] + eproj_ref[...])
    m = _silu(jnp.dot(m1, we2_ref[...],
                      preferred_element_type=jnp.float32) + be2_ref[...])
    u = _silu(jnp.dot(m, wx1_ref[...],
                      preferred_element_type=jnp.float32) + bx1_ref[...])
    w = jnp.tanh(jnp.sum(u * wx2_ref[...], axis=1, keepdims=True)
                 + bx2_ref[...])
    oh_sT = (lax.broadcasted_iota(jnp.int32, (N_LIG, EDGE_BLK), 0)
             == srcr_ref[...]).astype(jnp.float32)

    @pl.when(i == 0)
    def _():
        aggb_ref[...] = jnp.zeros_like(aggb_ref)
        xaggb_ref[...] = jnp.zeros_like(xaggb_ref)

    aggb_ref[...] += jnp.dot(oh_sT, m, preferred_element_type=jnp.float32)
    xaggb_ref[...] += jnp.dot(oh_sT, rel * w,
                              preferred_element_type=jnp.float32)
    en = _silu(jnp.dot(m, wed_ref[...],
                       preferred_element_type=jnp.float32) + bed_ref[...])
    pe1 = jax.nn.relu(jnp.dot(en, wq1_ref[...],
                              preferred_element_type=jnp.float32) + bq1_ref[...])
    pe_ref[...] = jnp.dot(pe1, wq2_ref[...],
                          preferred_element_type=jnp.float32) + bq2_ref[...]


def kernel(protein_node, protein_pos, protein_batch, ligand_node_pert,
           ligand_pos_pert, ligand_batch, ligand_edge_pert, ligand_edge_index,
           ligand_edge_batch, t, lab, params):
    p = params
    f32 = jnp.float32

    offset = jnp.linspace(0.0, float(NUM_TIMESTEPS), TIME_DIM)
    coeff = -0.5 / (offset[1] - offset[0]) ** 2
    te4 = jnp.exp(coeff * (t[:, None].astype(f32) - offset[None, :]) ** 2)
    ch = lab @ p['Wc1'] + p['bc1']
    mu = ch.mean(-1, keepdims=True)
    var = ch.var(-1, keepdims=True)
    ch = (ch - mu) / jnp.sqrt(var + 1e-5) * p['ln_g'] + p['ln_b']
    ce4 = jax.nn.gelu(ch) @ p['Wc2'] + p['bc2']

    we1 = p['We1']
    A = we1[0:NODE_DIM]
    B = we1[NODE_DIM:2 * NODE_DIM]
    vrow = we1[2 * NODE_DIM:2 * NODE_DIM + 1]
    R = we1[2 * NODE_DIM + 1:]

    nd_l = NODE_DIM - 1 - TIME_DIM - CLASS_EMB_DIM
    g4 = jnp.concatenate(
        [jnp.zeros((N_GRAPHS, nd_l), f32), te4, ce4,
         jnp.zeros((N_GRAPHS, 1), f32)], axis=1)
    wl_pad = jnp.concatenate(
        [p['W_lig_node'], jnp.zeros((p['W_lig_node'].shape[0],
                                     NODE_DIM - nd_l), f32)], axis=1)
    e127 = jnp.zeros((1, NODE_DIM), f32).at[0, NODE_DIM - 1].set(1.0)
    wprot_pad = jnp.concatenate(
        [p['W_prot_node'], jnp.zeros((p['W_prot_node'].shape[0], 1), f32)],
        axis=1)
    wprot_comb = wprot_pad @ B
    ed_l = EDGE_DIM - TIME_DIM - CLASS_EMB_DIM
    wle_comb = p['W_lig_edge'] @ R[0:ed_l]
    ge4 = jnp.concatenate([te4, ce4], axis=1) @ R[ed_l:]
    cknn = (p['W_prot_edge'][0] @ R)[None, :]
    be1 = p['be1'][None, :]

    all_pos = jnp.concatenate([protein_pos, ligand_pos_pert], 0)
    all_batch = jnp.concatenate([protein_batch, ligand_batch], 0)
    lb_col = ligand_batch[:, None].astype(jnp.int32)
    eb_col = ligand_edge_batch[:, None].astype(jnp.int32)

    pos16 = jnp.concatenate(
        [all_pos, jnp.zeros((N_ALL, TBL_W - NODE_DIM - 3), f32)], axis=1)
    tbl, hlig, hsa, eproj = pl.pallas_call(
        _embed_body,
        out_shape=[
            jax.ShapeDtypeStruct((N_ALL, TBL_W), f32),
            jax.ShapeDtypeStruct((N_LIG, NODE_DIM), f32),
            jax.ShapeDtypeStruct((N_LIG, NODE_DIM), f32),
            jax.ShapeDtypeStruct((E_LIG, NODE_DIM), f32),
        ],
    )(protein_node, wprot_comb, ligand_node_pert, wl_pad, lb_col, g4, e127,
      A, B, be1, ligand_edge_pert, wle_comb, eb_col, ge4, pos16)

    nbr = pl.pallas_call(
        _knn_body,
        out_shape=jax.ShapeDtypeStruct((N_LIG, K_NN), jnp.int32),
    )(ligand_pos_pert, lb_col, all_pos.T, all_batch[None, :].astype(jnp.int32))

    idx2 = nbr.reshape(GATHER_B // GATHER_CH, GATHER_CH)
    g = _sc_gather(tbl, idx2)

    wx2row = p['Wx2'][:, 0][None, :]
    bx2 = p['bx2'][None, :]

    src_l = ligand_edge_index[0].astype(jnp.int32)
    dst_l = ligand_edge_index[1].astype(jnp.int32)
    hdl = tbl[N_PROT:, 0:NODE_DIM]
    wspec = pl.BlockSpec((NODE_DIM, NODE_DIM), lambda i: (0, 0))
    rspec = pl.BlockSpec((1, NODE_DIM), lambda i: (0, 0))
    pe, aggb, xaggb = pl.pallas_call(
        _bond_body,
        grid=(N_EDGE_BLKS,),
        in_specs=[
            pl.BlockSpec((EDGE_BLK, 1), lambda i: (i, 0)),
            pl.BlockSpec((1, EDGE_BLK), lambda i: (0, i)),
            pl.BlockSpec((EDGE_BLK, 1), lambda i: (i, 0)),
            pl.BlockSpec((1, EDGE_BLK), lambda i: (0, i)),
            pl.BlockSpec((N_LIG, NODE_DIM), lambda i: (0, 0)),
            pl.BlockSpec((N_LIG, NODE_DIM), lambda i: (0, 0)),
            pl.BlockSpec((N_LIG, 3), lambda i: (0, 0)),
            pl.BlockSpec((EDGE_BLK, NODE_DIM), lambda i: (i, 0)),
            wspec, rspec, wspec, rspec,
            rspec, pl.BlockSpec((1, 1), lambda i: (0, 0)), rspec,
            pl.BlockSpec((NODE_DIM, EDGE_DIM), lambda i: (0, 0)),
            pl.BlockSpec((1, EDGE_DIM), lambda i: (0, 0)),
            pl.BlockSpec((EDGE_DIM, EDGE_DIM), lambda i: (0, 0)),
            pl.BlockSpec((1, EDGE_DIM), lambda i: (0, 0)),
            pl.BlockSpec((EDGE_DIM, 5), lambda i: (0, 0)),
            pl.BlockSpec((1, 5), lambda i: (0, 0)),
        ],
        out_specs=[
            pl.BlockSpec((EDGE_BLK, 5), lambda i: (i, 0)),
            pl.BlockSpec((N_LIG, NODE_DIM), lambda i: (0, 0)),
            pl.BlockSpec((N_LIG, 3), lambda i: (0, 0)),
        ],
        out_shape=[
            jax.ShapeDtypeStruct((E_LIG, 5), f32),
            jax.ShapeDtypeStruct((N_LIG, NODE_DIM), f32),
            jax.ShapeDtypeStruct((N_LIG, 3), f32),
        ],
    )(src_l[:, None], src_l[None, :], dst_l[:, None], dst_l[None, :],
      hsa, hdl, ligand_pos_pert, eproj,
      p['We2'], p['be2'][None, :], p['Wx1'], p['bx1'][None, :],
      wx2row, bx2, vrow,
      p['Wed'], p['bed'][None, :], p['Wq1'], p['bq1'][None, :],
      p['Wq2'], p['bq2'][None, :])

    wspec2 = pl.BlockSpec((NODE_DIM, NODE_DIM), lambda i: (0, 0))
    rspec2 = pl.BlockSpec((1, NODE_DIM), lambda i: (0, 0))
    pn, pp = pl.pallas_call(
        _knn_msg_body,
        grid=(N_LIG_BLKS,),
        in_specs=[
            pl.BlockSpec((LIG_BLK * K_NN, TBL_W), lambda i: (i, 0)),
            pl.BlockSpec((LIG_BLK, NODE_DIM), lambda i: (i, 0)),
            pl.BlockSpec((LIG_BLK, 3), lambda i: (i, 0)),
            pl.BlockSpec((LIG_BLK, K_NN), lambda i: (i, 0)),
            wspec2, rspec2, wspec2, rspec2, rspec2,
            pl.BlockSpec((1, 1), lambda i: (0, 0)),
            rspec2, rspec2,
            pl.BlockSpec((LIG_BLK, NODE_DIM), lambda i: (i, 0)),
            pl.BlockSpec((LIG_BLK, 3), lambda i: (i, 0)),
            pl.BlockSpec((LIG_BLK, NODE_DIM), lambda i: (i, 0)),
            wspec2, wspec2, rspec2, wspec2, rspec2,
            wspec2, rspec2,
            pl.BlockSpec((NODE_DIM, p['Wd2'].shape[1]), lambda i: (0, 0)),
            pl.BlockSpec((1, p['Wd2'].shape[1]), lambda i: (0, 0)),
        ],
        out_specs=[
            pl.BlockSpec((LIG_BLK, p['Wd2'].shape[1]), lambda i: (i, 0)),
            pl.BlockSpec((LIG_BLK, 3), lambda i: (i, 0)),
        ],
        out_shape=[
            jax.ShapeDtypeStruct((N_LIG, p['Wd2'].shape[1]), f32),
            jax.ShapeDtypeStruct((N_LIG, 3), f32),
        ],
    )(g, hsa, ligand_pos_pert, nbr, p['We2'], p['be2'][None, :],
      p['Wx1'], p['bx1'][None, :], wx2row, bx2, vrow, cknn,
      aggb, xaggb, hlig,
      p['Wh1'][0:NODE_DIM], p['Wh1'][NODE_DIM:], p['bh1'][None, :],
      p['Wh2'], p['bh2'][None, :], p['Wd1'], p['bd1'][None, :],
      p['Wd2'], p['bd2'][None, :])

    return pn, pp, pe


# speedup vs baseline: 1.3272x; 1.1792x over previous
import jax
import jax.numpy as jnp
from jax import lax
from jax.experimental import pallas as pl
from jax.experimental.pallas import tpu as pltpu
from jax.experimental.pallas import tpu_sc as plsc

N_PROT = 4096
N_LIG = 1024
N_ALL = N_PROT + N_LIG
E_LIG = 2048
N_GRAPHS = 4
NODE_DIM = 128
EDGE_DIM = 64
TIME_DIM = 16
CLASS_DIM = 8
CLASS_EMB_DIM = 16
NUM_TIMESTEPS = 1000
K_NN = 32

LIG_BLK = 128
N_LIG_BLKS = N_LIG // LIG_BLK
EDGE_BLK = 512
N_EDGE_BLKS = E_LIG // EDGE_BLK
TBL_W = 256

SC_NC = 2
SC_NS = 16
SC_NW = SC_NC * SC_NS
GATHER_B = N_LIG * K_NN
GATHER_CH = 128
GATHER_NCH = GATHER_B // (SC_NW * GATHER_CH)


def _silu(x):
    return x * jax.nn.sigmoid(x)


def _embed_body(prot_ref, wpc_ref, lig_ref, wl_ref, lb_ref, g4_ref, e127_ref,
                a_ref, b_ref, be1_ref, lep_ref, wle_ref, eb_ref, ge4_ref,
                pos16_ref, hd_ref, hlig_ref, hsa_ref, eproj_ref):
    oh_lb = (lax.broadcasted_iota(jnp.int32, (N_LIG, N_GRAPHS), 1)
             == lb_ref[...]).astype(jnp.float32)
    hl = (jnp.dot(lig_ref[...], wl_ref[...], preferred_element_type=jnp.float32)
          + jnp.dot(oh_lb, g4_ref[...], preferred_element_type=jnp.float32)
          + e127_ref[...])
    hlig_ref[...] = hl
    hsa_ref[...] = jnp.dot(hl, a_ref[...],
                           preferred_element_type=jnp.float32) + be1_ref[...]
    hd_ref[0:N_PROT, 0:NODE_DIM] = jnp.dot(
        prot_ref[...], wpc_ref[...], preferred_element_type=jnp.float32)
    hd_ref[N_PROT:N_ALL, 0:NODE_DIM] = jnp.dot(
        hl, b_ref[...], preferred_element_type=jnp.float32)
    hd_ref[:, NODE_DIM:TBL_W] = pos16_ref[...]
    oh_eb = (lax.broadcasted_iota(jnp.int32, (E_LIG, N_GRAPHS), 1)
             == eb_ref[...]).astype(jnp.float32)
    eproj_ref[...] = (
        jnp.dot(lep_ref[...], wle_ref[...], preferred_element_type=jnp.float32)
        + jnp.dot(oh_eb, ge4_ref[...], preferred_element_type=jnp.float32))


def _knn_body(ligpos_ref, lb_ref, posT_ref, ab_ref, nbr_ref):
    xl = ligpos_ref[...]
    posT = posT_ref[...]
    sq_all = jnp.sum(posT * posT, axis=0, keepdims=True)
    sq_lig = jnp.sum(xl * xl, axis=1, keepdims=True)
    d2 = sq_lig + sq_all - 2.0 * jnp.dot(
        xl, posT, preferred_element_type=jnp.float32)
    col = lax.broadcasted_iota(jnp.int32, (N_LIG, N_ALL), 1)
    same_batch = ab_ref[...] == lb_ref[...]
    row_gid = N_PROT + lax.broadcasted_iota(
        jnp.int32, (N_LIG, N_ALL), 0)
    inf = jnp.float32(jnp.inf)
    d2 = jnp.where(jnp.logical_and(same_batch, col != row_gid), d2, inf)
    kcol = lax.broadcasted_iota(jnp.int32, (N_LIG, K_NN), 1)

    def step(k, carry):
        d2c, acc = carry
        mn = jnp.min(d2c, axis=1, keepdims=True)
        idx = jnp.min(jnp.where(d2c == mn, col, N_ALL), axis=1, keepdims=True)
        acc = jnp.where(kcol == k, idx, acc)
        return jnp.where(col == idx, inf, d2c), acc

    _, nbr = lax.fori_loop(
        0, K_NN, step,
        (d2, jnp.zeros((N_LIG, K_NN), jnp.int32)), unroll=2)
    nbr_ref[...] = nbr


def _sc_gather_body(tbl_ref, idx_ref, out_ref, idx_v, rows_v, sem):
    wid = lax.axis_index("s") * SC_NC + lax.axis_index("c")
    base_ch = wid * GATHER_NCH
    pltpu.sync_copy(idx_ref.at[pl.ds(base_ch, GATHER_NCH)], idx_v)
    for c in range(GATHER_NCH):
        pltpu.async_copy(tbl_ref.at[idx_v.at[c]], rows_v, sem).wait()
        pltpu.sync_copy(
            rows_v, out_ref.at[pl.ds((base_ch + c) * GATHER_CH, GATHER_CH)])


def _sc_gather(tbl, idx2):
    run = pl.kernel(
        _sc_gather_body,
        mesh=plsc.VectorSubcoreMesh(core_axis_name="c", subcore_axis_name="s"),
        out_type=jax.ShapeDtypeStruct((GATHER_B, TBL_W), jnp.float32),
        scratch_types=[
            pltpu.VMEM((GATHER_NCH, GATHER_CH), jnp.int32),
            pltpu.VMEM((GATHER_CH, TBL_W), jnp.float32),
            pltpu.SemaphoreType.DMA,
        ],
    )
    return run(tbl, idx2)


def _knn_msg_body(g_ref, hsa_ref, xl_ref, nbr_ref, we2_ref, be2_ref,
                  wx1_ref, bx1_ref, wx2_ref, bx2_ref, v_ref, ck_ref,
                  aggb_ref, xaggb_ref, hlig_ref, wh1a_ref, wh1b_ref, bh1_ref,
                  wh2_ref, bh2_ref, wd1_ref, bd1_ref, wd2_ref, bd2_ref,
                  pn_ref, pp_ref):
    g = g_ref[...]
    d = g[:, 0:NODE_DIM].reshape(LIG_BLK, K_NN, NODE_DIM)
    pnb = g[:, NODE_DIM:NODE_DIM + 3].reshape(LIG_BLK, K_NN, 3)
    xl = xl_ref[...]
    rel = xl[:, None, :] - pnb
    d2 = jnp.sum(rel * rel, axis=2, keepdims=True)
    m1 = _silu(hsa_ref[...][:, None, :] + d
               + d2 * v_ref[...][None] + ck_ref[...][None])
    m1f = m1.reshape(LIG_BLK * K_NN, NODE_DIM)
    m = _silu(jnp.dot(m1f, we2_ref[...],
                      preferred_element_type=jnp.float32) + be2_ref[...])
    u = _silu(jnp.dot(m, wx1_ref[...],
                      preferred_element_type=jnp.float32) + bx1_ref[...])
    w = jnp.tanh(jnp.sum(u * wx2_ref[...], axis=1, keepdims=True)
                 + bx2_ref[...])
    mask = (nbr_ref[...] < N_PROT).astype(jnp.float32)
    m3 = m.reshape(LIG_BLK, K_NN, NODE_DIM) * mask[:, :, None]
    agg = jnp.sum(m3, axis=1) + aggb_ref[...]
    wm = (w.reshape(LIG_BLK, K_NN, 1) * mask[:, :, None])
    xagg = jnp.sum(rel * wm, axis=1) + xaggb_ref[...]
    hl = hlig_ref[...]
    t1 = _silu(jnp.dot(hl, wh1a_ref[...], preferred_element_type=jnp.float32)
               + jnp.dot(agg, wh1b_ref[...],
                         preferred_element_type=jnp.float32) + bh1_ref[...])
    hn = hl + jnp.dot(t1, wh2_ref[...],
                      preferred_element_type=jnp.float32) + bh2_ref[...]
    pn1 = jax.nn.relu(jnp.dot(hn, wd1_ref[...],
                              preferred_element_type=jnp.float32) + bd1_ref[...])
    pn_ref[...] = jnp.dot(pn1, wd2_ref[...],
                          preferred_element_type=jnp.float32) + bd2_ref[...]
    pp_ref[...] = xl + xagg / K_NN


def _bond_body(srcc_ref, srcr_ref, dstc_ref, dstr_ref, hsa_ref, hdl_ref,
                    xlig_ref, eproj_ref, we2_ref, be2_ref, wx1_ref, bx1_ref,
                    wx2_ref, bx2_ref, v_ref, wed_ref, bed_ref, wq1_ref,
                    bq1_ref, wq2_ref, bq2_ref, pe_ref, aggb_ref, xaggb_ref):
    i = pl.program_id(0)
    oh_s = (lax.broadcasted_iota(jnp.int32, (EDGE_BLK, N_LIG), 1)
            == srcc_ref[...]).astype(jnp.float32)
    oh_d = (lax.broadcasted_iota(jnp.int32, (EDGE_BLK, N_LIG), 1)
            == dstc_ref[...]).astype(jnp.float32)
    hs = jnp.dot(oh_s, hsa_ref[...], preferred_element_type=jnp.float32)
    hd = jnp.dot(oh_d, hdl_ref[...], preferred_element_type=jnp.float32)
    xs = jnp.dot(oh_s, xlig_ref[...], preferred_element_type=jnp.float32)
    xd = jnp.dot(oh_d, xlig_ref[...], preferred_element_type=jnp.float32)
    rel = xs - xd
    d2 = jnp.sum(rel * rel, axis=1, keepdims=True)
    m1 = _silu(hs + hd + d2 * v_ref[...] + eproj_ref[...])
    m = _silu(jnp.dot(m1, we2_ref[...],
                      preferred_element_type=jnp.float32) + be2_ref[...])
    u = _silu(jnp.dot(m, wx1_ref[...],
                      preferred_element_type=jnp.float32) + bx1_ref[...])
    w = jnp.tanh(jnp.sum(u * wx2_ref[...], axis=1, keepdims=True)
                 + bx2_ref[...])
    oh_sT = (lax.broadcasted_iota(jnp.int32, (N_LIG, EDGE_BLK), 0)
             == srcr_ref[...]).astype(jnp.float32)

    @pl.when(i == 0)
    def _():
        aggb_ref[...] = jnp.zeros_like(aggb_ref)
        xaggb_ref[...] = jnp.zeros_like(xaggb_ref)

    aggb_ref[...] += jnp.dot(oh_sT, m, preferred_element_type=jnp.float32)
    xaggb_ref[...] += jnp.dot(oh_sT, rel * w,
                              preferred_element_type=jnp.float32)
    en = _silu(jnp.dot(m, wed_ref[...],
                       preferred_element_type=jnp.float32) + bed_ref[...])
    pe1 = jax.nn.relu(jnp.dot(en, wq1_ref[...],
                              preferred_element_type=jnp.float32) + bq1_ref[...])
    pe_ref[...] = jnp.dot(pe1, wq2_ref[...],
                          preferred_element_type=jnp.float32) + bq2_ref[...]


def kernel(protein_node, protein_pos, protein_batch, ligand_node_pert,
           ligand_pos_pert, ligand_batch, ligand_edge_pert, ligand_edge_index,
           ligand_edge_batch, t, lab, params):
    p = params
    f32 = jnp.float32

    offset = jnp.linspace(0.0, float(NUM_TIMESTEPS), TIME_DIM)
    coeff = -0.5 / (offset[1] - offset[0]) ** 2
    te4 = jnp.exp(coeff * (t[:, None].astype(f32) - offset[None, :]) ** 2)
    ch = lab @ p['Wc1'] + p['bc1']
    mu = ch.mean(-1, keepdims=True)
    var = ch.var(-1, keepdims=True)
    ch = (ch - mu) / jnp.sqrt(var + 1e-5) * p['ln_g'] + p['ln_b']
    ce4 = jax.nn.gelu(ch) @ p['Wc2'] + p['bc2']

    we1 = p['We1']
    A = we1[0:NODE_DIM]
    B = we1[NODE_DIM:2 * NODE_DIM]
    vrow = we1[2 * NODE_DIM:2 * NODE_DIM + 1]
    R = we1[2 * NODE_DIM + 1:]

    nd_l = NODE_DIM - 1 - TIME_DIM - CLASS_EMB_DIM
    g4 = jnp.concatenate(
        [jnp.zeros((N_GRAPHS, nd_l), f32), te4, ce4,
         jnp.zeros((N_GRAPHS, 1), f32)], axis=1)
    wl_pad = jnp.concatenate(
        [p['W_lig_node'], jnp.zeros((p['W_lig_node'].shape[0],
                                     NODE_DIM - nd_l), f32)], axis=1)
    e127 = jnp.zeros((1, NODE_DIM), f32).at[0, NODE_DIM - 1].set(1.0)
    wprot_pad = jnp.concatenate(
        [p['W_prot_node'], jnp.zeros((p['W_prot_node'].shape[0], 1), f32)],
        axis=1)
    wprot_comb = wprot_pad @ B
    ed_l = EDGE_DIM - TIME_DIM - CLASS_EMB_DIM
    wle_comb = p['W_lig_edge'] @ R[0:ed_l]
    ge4 = jnp.concatenate([te4, ce4], axis=1) @ R[ed_l:]
    cknn = (p['W_prot_edge'][0] @ R)[None, :]
    be1 = p['be1'][None, :]

    all_pos = jnp.concatenate([protein_pos, ligand_pos_pert], 0)
    all_batch = jnp.concatenate([protein_batch, ligand_batch], 0)
    lb_col = ligand_batch[:, None].astype(jnp.int32)
    eb_col = ligand_edge_batch[:, None].astype(jnp.int32)

    pos16 = jnp.concatenate(
        [all_pos, jnp.zeros((N_ALL, TBL_W - NODE_DIM - 3), f32)], axis=1)
    tbl, hlig, hsa, eproj = pl.pallas_call(
        _embed_body,
        out_shape=[
            jax.ShapeDtypeStruct((N_ALL, TBL_W), f32),
            jax.ShapeDtypeStruct((N_LIG, NODE_DIM), f32),
            jax.ShapeDtypeStruct((N_LIG, NODE_DIM), f32),
            jax.ShapeDtypeStruct((E_LIG, NODE_DIM), f32),
        ],
    )(protein_node, wprot_comb, ligand_node_pert, wl_pad, lb_col, g4, e127,
      A, B, be1, ligand_edge_pert, wle_comb, eb_col, ge4, pos16)

    nbr = pl.pallas_call(
        _knn_body,
        out_shape=jax.ShapeDtypeStruct((N_LIG, K_NN), jnp.int32),
    )(ligand_pos_pert, lb_col, all_pos.T, all_batch[None, :].astype(jnp.int32))

    idx2 = nbr.reshape(GATHER_B // GATHER_CH, GATHER_CH)
    g = _sc_gather(tbl, idx2)

    wx2row = p['Wx2'][:, 0][None, :]
    bx2 = p['bx2'][None, :]

    src_l = ligand_edge_index[0].astype(jnp.int32)
    dst_l = ligand_edge_index[1].astype(jnp.int32)
    hdl = tbl[N_PROT:, 0:NODE_DIM]
    wspec = pl.BlockSpec((NODE_DIM, NODE_DIM), lambda i: (0, 0))
    rspec = pl.BlockSpec((1, NODE_DIM), lambda i: (0, 0))
    pe, aggb, xaggb = pl.pallas_call(
        _bond_body,
        grid=(N_EDGE_BLKS,),
        in_specs=[
            pl.BlockSpec((EDGE_BLK, 1), lambda i: (i, 0)),
            pl.BlockSpec((1, EDGE_BLK), lambda i: (0, i)),
            pl.BlockSpec((EDGE_BLK, 1), lambda i: (i, 0)),
            pl.BlockSpec((1, EDGE_BLK), lambda i: (0, i)),
            pl.BlockSpec((N_LIG, NODE_DIM), lambda i: (0, 0)),
            pl.BlockSpec((N_LIG, NODE_DIM), lambda i: (0, 0)),
            pl.BlockSpec((N_LIG, 3), lambda i: (0, 0)),
            pl.BlockSpec((EDGE_BLK, NODE_DIM), lambda i: (i, 0)),
            wspec, rspec, wspec, rspec,
            rspec, pl.BlockSpec((1, 1), lambda i: (0, 0)), rspec,
            pl.BlockSpec((NODE_DIM, EDGE_DIM), lambda i: (0, 0)),
            pl.BlockSpec((1, EDGE_DIM), lambda i: (0, 0)),
            pl.BlockSpec((EDGE_DIM, EDGE_DIM), lambda i: (0, 0)),
            pl.BlockSpec((1, EDGE_DIM), lambda i: (0, 0)),
            pl.BlockSpec((EDGE_DIM, 5), lambda i: (0, 0)),
            pl.BlockSpec((1, 5), lambda i: (0, 0)),
        ],
        out_specs=[
            pl.BlockSpec((EDGE_BLK, 5), lambda i: (i, 0)),
            pl.BlockSpec((N_LIG, NODE_DIM), lambda i: (0, 0)),
            pl.BlockSpec((N_LIG, 3), lambda i: (0, 0)),
        ],
        out_shape=[
            jax.ShapeDtypeStruct((E_LIG, 5), f32),
            jax.ShapeDtypeStruct((N_LIG, NODE_DIM), f32),
            jax.ShapeDtypeStruct((N_LIG, 3), f32),
        ],
    )(src_l[:, None], src_l[None, :], dst_l[:, None], dst_l[None, :],
      hsa, hdl, ligand_pos_pert, eproj,
      p['We2'], p['be2'][None, :], p['Wx1'], p['bx1'][None, :],
      wx2row, bx2, vrow,
      p['Wed'], p['bed'][None, :], p['Wq1'], p['bq1'][None, :],
      p['Wq2'], p['bq2'][None, :])

    wspec2 = pl.BlockSpec((NODE_DIM, NODE_DIM), lambda i: (0, 0))
    rspec2 = pl.BlockSpec((1, NODE_DIM), lambda i: (0, 0))
    pn, pp = pl.pallas_call(
        _knn_msg_body,
        grid=(N_LIG_BLKS,),
        in_specs=[
            pl.BlockSpec((LIG_BLK * K_NN, TBL_W), lambda i: (i, 0)),
            pl.BlockSpec((LIG_BLK, NODE_DIM), lambda i: (i, 0)),
            pl.BlockSpec((LIG_BLK, 3), lambda i: (i, 0)),
            pl.BlockSpec((LIG_BLK, K_NN), lambda i: (i, 0)),
            wspec2, rspec2, wspec2, rspec2, rspec2,
            pl.BlockSpec((1, 1), lambda i: (0, 0)),
            rspec2, rspec2,
            pl.BlockSpec((LIG_BLK, NODE_DIM), lambda i: (i, 0)),
            pl.BlockSpec((LIG_BLK, 3), lambda i: (i, 0)),
            pl.BlockSpec((LIG_BLK, NODE_DIM), lambda i: (i, 0)),
            wspec2, wspec2, rspec2, wspec2, rspec2,
            wspec2, rspec2,
            pl.BlockSpec((NODE_DIM, p['Wd2'].shape[1]), lambda i: (0, 0)),
            pl.BlockSpec((1, p['Wd2'].shape[1]), lambda i: (0, 0)),
        ],
        out_specs=[
            pl.BlockSpec((LIG_BLK, p['Wd2'].shape[1]), lambda i: (i, 0)),
            pl.BlockSpec((LIG_BLK, 3), lambda i: (i, 0)),
        ],
        out_shape=[
            jax.ShapeDtypeStruct((N_LIG, p['Wd2'].shape[1]), f32),
            jax.ShapeDtypeStruct((N_LIG, 3), f32),
        ],
    )(g, hsa, ligand_pos_pert, nbr, p['We2'], p['be2'][None, :],
      p['Wx1'], p['bx1'][None, :], wx2row, bx2, vrow, cknn,
      aggb, xaggb, hlig,
      p['Wh1'][0:NODE_DIM], p['Wh1'][NODE_DIM:], p['bh1'][None, :],
      p['Wh2'], p['bh2'][None, :], p['Wd1'], p['bd1'][None, :],
      p['Wd2'], p['bd2'][None, :])

    return pn, pp, pe


# speedup vs baseline: 1.4334x; 1.0800x over previous
import jax
import jax.numpy as jnp
from jax import lax
from jax.experimental import pallas as pl
from jax.experimental.pallas import tpu as pltpu
from jax.experimental.pallas import tpu_sc as plsc

N_PROT = 4096
N_LIG = 1024
N_ALL = N_PROT + N_LIG
E_LIG = 2048
N_GRAPHS = 4
NODE_DIM = 128
EDGE_DIM = 64
TIME_DIM = 16
CLASS_DIM = 8
CLASS_EMB_DIM = 16
NUM_TIMESTEPS = 1000
K_NN = 32

LIG_BLK = 128
N_LIG_BLKS = N_LIG // LIG_BLK
EDGE_BLK = 512
N_EDGE_BLKS = E_LIG // EDGE_BLK
TBL_W = 256

SC_NC = 2
SC_NS = 16
SC_NW = SC_NC * SC_NS
GATHER_B = N_LIG * K_NN
GATHER_CH = 128
GATHER_NCH = GATHER_B // (SC_NW * GATHER_CH)


def _silu(x):
    return x * jax.nn.sigmoid(x)


def _embed_body(prot_ref, wpc_ref, lig_ref, wl_ref, lb_ref, g4_ref, e127_ref,
                a_ref, b_ref, be1_ref, lep_ref, wle_ref, eb_ref, ge4_ref,
                pos16_ref, hd_ref, hlig_ref, hsa_ref, eproj_ref):
    oh_lb = (lax.broadcasted_iota(jnp.int32, (N_LIG, N_GRAPHS), 1)
             == lb_ref[...]).astype(jnp.float32)
    hl = (jnp.dot(lig_ref[...], wl_ref[...], preferred_element_type=jnp.float32)
          + jnp.dot(oh_lb, g4_ref[...], preferred_element_type=jnp.float32)
          + e127_ref[...])
    hlig_ref[...] = hl
    hsa_ref[...] = jnp.dot(hl, a_ref[...],
                           preferred_element_type=jnp.float32) + be1_ref[...]
    hd_ref[0:N_PROT, 0:NODE_DIM] = jnp.dot(
        prot_ref[...], wpc_ref[...], preferred_element_type=jnp.float32)
    hd_ref[N_PROT:N_ALL, 0:NODE_DIM] = jnp.dot(
        hl, b_ref[...], preferred_element_type=jnp.float32)
    hd_ref[:, NODE_DIM:TBL_W] = pos16_ref[...]
    oh_eb = (lax.broadcasted_iota(jnp.int32, (E_LIG, N_GRAPHS), 1)
             == eb_ref[...]).astype(jnp.float32)
    eproj_ref[...] = (
        jnp.dot(lep_ref[...], wle_ref[...], preferred_element_type=jnp.float32)
        + jnp.dot(oh_eb, ge4_ref[...], preferred_element_type=jnp.float32))


def _knn_body(ligpos_ref, lb_ref, posT_ref, ab_ref, nbr_ref):
    xl = ligpos_ref[...]
    posT = posT_ref[...]
    sq_all = jnp.sum(posT * posT, axis=0, keepdims=True)
    sq_lig = jnp.sum(xl * xl, axis=1, keepdims=True)
    d2 = sq_lig + sq_all - 2.0 * jnp.dot(
        xl, posT, preferred_element_type=jnp.float32)
    col = lax.broadcasted_iota(jnp.int32, (N_LIG, N_ALL), 1)
    same_batch = ab_ref[...] == lb_ref[...]
    row_gid = N_PROT + lax.broadcasted_iota(
        jnp.int32, (N_LIG, N_ALL), 0)
    inf = jnp.float32(jnp.inf)
    d2 = jnp.where(jnp.logical_and(same_batch, col != row_gid), d2, inf)
    kcol = lax.broadcasted_iota(jnp.int32, (N_LIG, K_NN), 1)

    def step(k, carry):
        d2c, acc = carry
        mn = jnp.min(d2c, axis=1, keepdims=True)
        idx = jnp.min(jnp.where(d2c == mn, col, N_ALL), axis=1, keepdims=True)
        acc = jnp.where(kcol == k, idx, acc)
        return jnp.where(col == idx, inf, d2c), acc

    _, nbr = lax.fori_loop(
        0, K_NN, step,
        (d2, jnp.zeros((N_LIG, K_NN), jnp.int32)), unroll=4)
    nbr_ref[...] = nbr


def _sc_gather_body(tbl_ref, idx_ref, out_ref, idx_v, rows_v, sem):
    wid = lax.axis_index("s") * SC_NC + lax.axis_index("c")
    base_ch = wid * GATHER_NCH
    pltpu.sync_copy(idx_ref.at[pl.ds(base_ch, GATHER_NCH)], idx_v)
    for c in range(GATHER_NCH):
        pltpu.async_copy(tbl_ref.at[idx_v.at[c]], rows_v, sem).wait()
        pltpu.sync_copy(
            rows_v, out_ref.at[pl.ds((base_ch + c) * GATHER_CH, GATHER_CH)])


def _sc_gather(tbl, idx2):
    run = pl.kernel(
        _sc_gather_body,
        mesh=plsc.VectorSubcoreMesh(core_axis_name="c", subcore_axis_name="s"),
        out_type=jax.ShapeDtypeStruct((GATHER_B, TBL_W), jnp.float32),
        scratch_types=[
            pltpu.VMEM((GATHER_NCH, GATHER_CH), jnp.int32),
            pltpu.VMEM((GATHER_CH, TBL_W), jnp.float32),
            pltpu.SemaphoreType.DMA,
        ],
    )
    return run(tbl, idx2)


def _knn_msg_body(g_ref, hsa_ref, xl_ref, nbr_ref, we2_ref, be2_ref,
                  wx1_ref, bx1_ref, wx2_ref, bx2_ref, v_ref, ck_ref,
                  aggb_ref, xaggb_ref, hlig_ref, wh1a_ref, wh1b_ref, bh1_ref,
                  wh2_ref, bh2_ref, wd1_ref, bd1_ref, wd2_ref, bd2_ref,
                  pn_ref, pp_ref):
    g = g_ref[...]
    d = g[:, 0:NODE_DIM].reshape(LIG_BLK, K_NN, NODE_DIM)
    pnb = g[:, NODE_DIM:NODE_DIM + 3].reshape(LIG_BLK, K_NN, 3)
    xl = xl_ref[...]
    rel = xl[:, None, :] - pnb
    d2 = jnp.sum(rel * rel, axis=2, keepdims=True)
    m1 = _silu(hsa_ref[...][:, None, :] + d
               + d2 * v_ref[...][None] + ck_ref[...][None])
    m1f = m1.reshape(LIG_BLK * K_NN, NODE_DIM)
    m = _silu(jnp.dot(m1f, we2_ref[...],
                      preferred_element_type=jnp.float32) + be2_ref[...])
    u = _silu(jnp.dot(m, wx1_ref[...],
                      preferred_element_type=jnp.float32) + bx1_ref[...])
    w = jnp.tanh(jnp.sum(u * wx2_ref[...], axis=1, keepdims=True)
                 + bx2_ref[...])
    mask = (nbr_ref[...] < N_PROT).astype(jnp.float32)
    m3 = m.reshape(LIG_BLK, K_NN, NODE_DIM) * mask[:, :, None]
    agg = jnp.sum(m3, axis=1) + aggb_ref[...]
    wm = (w.reshape(LIG_BLK, K_NN, 1) * mask[:, :, None])
    xagg = jnp.sum(rel * wm, axis=1) + xaggb_ref[...]
    hl = hlig_ref[...]
    t1 = _silu(jnp.dot(hl, wh1a_ref[...], preferred_element_type=jnp.float32)
               + jnp.dot(agg, wh1b_ref[...],
                         preferred_element_type=jnp.float32) + bh1_ref[...])
    hn = hl + jnp.dot(t1, wh2_ref[...],
                      preferred_element_type=jnp.float32) + bh2_ref[...]
    pn1 = jax.nn.relu(jnp.dot(hn, wd1_ref[...],
                              preferred_element_type=jnp.float32) + bd1_ref[...])
    pn_ref[...] = jnp.dot(pn1, wd2_ref[...],
                          preferred_element_type=jnp.float32) + bd2_ref[...]
    pp_ref[...] = xl + xagg / K_NN


def _bond_body(srcc_ref, srcr_ref, dstc_ref, dstr_ref, hsa_ref, hdl_ref,
                    xlig_ref, eproj_ref, we2_ref, be2_ref, wx1_ref, bx1_ref,
                    wx2_ref, bx2_ref, v_ref, wed_ref, bed_ref, wq1_ref,
                    bq1_ref, wq2_ref, bq2_ref, pe_ref, aggb_ref, xaggb_ref):
    i = pl.program_id(0)
    oh_s = (lax.broadcasted_iota(jnp.int32, (EDGE_BLK, N_LIG), 1)
            == srcc_ref[...]).astype(jnp.float32)
    oh_d = (lax.broadcasted_iota(jnp.int32, (EDGE_BLK, N_LIG), 1)
            == dstc_ref[...]).astype(jnp.float32)
    hs = jnp.dot(oh_s, hsa_ref[...], preferred_element_type=jnp.float32)
    hd = jnp.dot(oh_d, hdl_ref[...], preferred_element_type=jnp.float32)
    xs = jnp.dot(oh_s, xlig_ref[...], preferred_element_type=jnp.float32)
    xd = jnp.dot(oh_d, xlig_ref[...], preferred_element_type=jnp.float32)
    rel = xs - xd
    d2 = jnp.sum(rel * rel, axis=1, keepdims=True)
    m1 = _silu(hs + hd + d2 * v_ref[...] + eproj_ref[...])
    m = _silu(jnp.dot(m1, we2_ref[...],
                      preferred_element_type=jnp.float32) + be2_ref[...])
    u = _silu(jnp.dot(m, wx1_ref[...],
                      preferred_element_type=jnp.float32) + bx1_ref[...])
    w = jnp.tanh(jnp.sum(u * wx2_ref[...], axis=1, keepdims=True)
                 + bx2_ref[...])
    oh_sT = (lax.broadcasted_iota(jnp.int32, (N_LIG, EDGE_BLK), 0)
             == srcr_ref[...]).astype(jnp.float32)

    @pl.when(i == 0)
    def _():
        aggb_ref[...] = jnp.zeros_like(aggb_ref)
        xaggb_ref[...] = jnp.zeros_like(xaggb_ref)

    aggb_ref[...] += jnp.dot(oh_sT, m, preferred_element_type=jnp.float32)
    xaggb_ref[...] += jnp.dot(oh_sT, rel * w,
                              preferred_element_type=jnp.float32)
    en = _silu(jnp.dot(m, wed_ref[...],
                       preferred_element_type=jnp.float32) + bed_ref[...])
    pe1 = jax.nn.relu(jnp.dot(en, wq1_ref[...],
                              preferred_element_type=jnp.float32) + bq1_ref[...])
    pe_ref[...] = jnp.dot(pe1, wq2_ref[...],
                          preferred_element_type=jnp.float32) + bq2_ref[...]


def kernel(protein_node, protein_pos, protein_batch, ligand_node_pert,
           ligand_pos_pert, ligand_batch, ligand_edge_pert, ligand_edge_index,
           ligand_edge_batch, t, lab, params):
    p = params
    f32 = jnp.float32

    offset = jnp.linspace(0.0, float(NUM_TIMESTEPS), TIME_DIM)
    coeff = -0.5 / (offset[1] - offset[0]) ** 2
    te4 = jnp.exp(coeff * (t[:, None].astype(f32) - offset[None, :]) ** 2)
    ch = lab @ p['Wc1'] + p['bc1']
    mu = ch.mean(-1, keepdims=True)
    var = ch.var(-1, keepdims=True)
    ch = (ch - mu) / jnp.sqrt(var + 1e-5) * p['ln_g'] + p['ln_b']
    ce4 = jax.nn.gelu(ch) @ p['Wc2'] + p['bc2']

    we1 = p['We1']
    A = we1[0:NODE_DIM]
    B = we1[NODE_DIM:2 * NODE_DIM]
    vrow = we1[2 * NODE_DIM:2 * NODE_DIM + 1]
    R = we1[2 * NODE_DIM + 1:]

    nd_l = NODE_DIM - 1 - TIME_DIM - CLASS_EMB_DIM
    g4 = jnp.concatenate(
        [jnp.zeros((N_GRAPHS, nd_l), f32), te4, ce4,
         jnp.zeros((N_GRAPHS, 1), f32)], axis=1)
    wl_pad = jnp.concatenate(
        [p['W_lig_node'], jnp.zeros((p['W_lig_node'].shape[0],
                                     NODE_DIM - nd_l), f32)], axis=1)
    e127 = jnp.zeros((1, NODE_DIM), f32).at[0, NODE_DIM - 1].set(1.0)
    wprot_pad = jnp.concatenate(
        [p['W_prot_node'], jnp.zeros((p['W_prot_node'].shape[0], 1), f32)],
        axis=1)
    wprot_comb = wprot_pad @ B
    ed_l = EDGE_DIM - TIME_DIM - CLASS_EMB_DIM
    wle_comb = p['W_lig_edge'] @ R[0:ed_l]
    ge4 = jnp.concatenate([te4, ce4], axis=1) @ R[ed_l:]
    cknn = (p['W_prot_edge'][0] @ R)[None, :]
    be1 = p['be1'][None, :]

    all_pos = jnp.concatenate([protein_pos, ligand_pos_pert], 0)
    all_batch = jnp.concatenate([protein_batch, ligand_batch], 0)
    lb_col = ligand_batch[:, None].astype(jnp.int32)
    eb_col = ligand_edge_batch[:, None].astype(jnp.int32)

    pos16 = jnp.concatenate(
        [all_pos, jnp.zeros((N_ALL, TBL_W - NODE_DIM - 3), f32)], axis=1)
    tbl, hlig, hsa, eproj = pl.pallas_call(
        _embed_body,
        out_shape=[
            jax.ShapeDtypeStruct((N_ALL, TBL_W), f32),
            jax.ShapeDtypeStruct((N_LIG, NODE_DIM), f32),
            jax.ShapeDtypeStruct((N_LIG, NODE_DIM), f32),
            jax.ShapeDtypeStruct((E_LIG, NODE_DIM), f32),
        ],
    )(protein_node, wprot_comb, ligand_node_pert, wl_pad, lb_col, g4, e127,
      A, B, be1, ligand_edge_pert, wle_comb, eb_col, ge4, pos16)

    nbr = pl.pallas_call(
        _knn_body,
        out_shape=jax.ShapeDtypeStruct((N_LIG, K_NN), jnp.int32),
    )(ligand_pos_pert, lb_col, all_pos.T, all_batch[None, :].astype(jnp.int32))

    idx2 = nbr.reshape(GATHER_B // GATHER_CH, GATHER_CH)
    g = _sc_gather(tbl, idx2)

    wx2row = p['Wx2'][:, 0][None, :]
    bx2 = p['bx2'][None, :]

    src_l = ligand_edge_index[0].astype(jnp.int32)
    dst_l = ligand_edge_index[1].astype(jnp.int32)
    hdl = tbl[N_PROT:, 0:NODE_DIM]
    wspec = pl.BlockSpec((NODE_DIM, NODE_DIM), lambda i: (0, 0))
    rspec = pl.BlockSpec((1, NODE_DIM), lambda i: (0, 0))
    pe, aggb, xaggb = pl.pallas_call(
        _bond_body,
        grid=(N_EDGE_BLKS,),
        in_specs=[
            pl.BlockSpec((EDGE_BLK, 1), lambda i: (i, 0)),
            pl.BlockSpec((1, EDGE_BLK), lambda i: (0, i)),
            pl.BlockSpec((EDGE_BLK, 1), lambda i: (i, 0)),
            pl.BlockSpec((1, EDGE_BLK), lambda i: (0, i)),
            pl.BlockSpec((N_LIG, NODE_DIM), lambda i: (0, 0)),
            pl.BlockSpec((N_LIG, NODE_DIM), lambda i: (0, 0)),
            pl.BlockSpec((N_LIG, 3), lambda i: (0, 0)),
            pl.BlockSpec((EDGE_BLK, NODE_DIM), lambda i: (i, 0)),
            wspec, rspec, wspec, rspec,
            rspec, pl.BlockSpec((1, 1), lambda i: (0, 0)), rspec,
            pl.BlockSpec((NODE_DIM, EDGE_DIM), lambda i: (0, 0)),
            pl.BlockSpec((1, EDGE_DIM), lambda i: (0, 0)),
            pl.BlockSpec((EDGE_DIM, EDGE_DIM), lambda i: (0, 0)),
            pl.BlockSpec((1, EDGE_DIM), lambda i: (0, 0)),
            pl.BlockSpec((EDGE_DIM, 5), lambda i: (0, 0)),
            pl.BlockSpec((1, 5), lambda i: (0, 0)),
        ],
        out_specs=[
            pl.BlockSpec((EDGE_BLK, 5), lambda i: (i, 0)),
            pl.BlockSpec((N_LIG, NODE_DIM), lambda i: (0, 0)),
            pl.BlockSpec((N_LIG, 3), lambda i: (0, 0)),
        ],
        out_shape=[
            jax.ShapeDtypeStruct((E_LIG, 5), f32),
            jax.ShapeDtypeStruct((N_LIG, NODE_DIM), f32),
            jax.ShapeDtypeStruct((N_LIG, 3), f32),
        ],
    )(src_l[:, None], src_l[None, :], dst_l[:, None], dst_l[None, :],
      hsa, hdl, ligand_pos_pert, eproj,
      p['We2'], p['be2'][None, :], p['Wx1'], p['bx1'][None, :],
      wx2row, bx2, vrow,
      p['Wed'], p['bed'][None, :], p['Wq1'], p['bq1'][None, :],
      p['Wq2'], p['bq2'][None, :])

    wspec2 = pl.BlockSpec((NODE_DIM, NODE_DIM), lambda i: (0, 0))
    rspec2 = pl.BlockSpec((1, NODE_DIM), lambda i: (0, 0))
    pn, pp = pl.pallas_call(
        _knn_msg_body,
        grid=(N_LIG_BLKS,),
        in_specs=[
            pl.BlockSpec((LIG_BLK * K_NN, TBL_W), lambda i: (i, 0)),
            pl.BlockSpec((LIG_BLK, NODE_DIM), lambda i: (i, 0)),
            pl.BlockSpec((LIG_BLK, 3), lambda i: (i, 0)),
            pl.BlockSpec((LIG_BLK, K_NN), lambda i: (i, 0)),
            wspec2, rspec2, wspec2, rspec2, rspec2,
            pl.BlockSpec((1, 1), lambda i: (0, 0)),
            rspec2, rspec2,
            pl.BlockSpec((LIG_BLK, NODE_DIM), lambda i: (i, 0)),
            pl.BlockSpec((LIG_BLK, 3), lambda i: (i, 0)),
            pl.BlockSpec((LIG_BLK, NODE_DIM), lambda i: (i, 0)),
            wspec2, wspec2, rspec2, wspec2, rspec2,
            wspec2, rspec2,
            pl.BlockSpec((NODE_DIM, p['Wd2'].shape[1]), lambda i: (0, 0)),
            pl.BlockSpec((1, p['Wd2'].shape[1]), lambda i: (0, 0)),
        ],
        out_specs=[
            pl.BlockSpec((LIG_BLK, p['Wd2'].shape[1]), lambda i: (i, 0)),
            pl.BlockSpec((LIG_BLK, 3), lambda i: (i, 0)),
        ],
        out_shape=[
            jax.ShapeDtypeStruct((N_LIG, p['Wd2'].shape[1]), f32),
            jax.ShapeDtypeStruct((N_LIG, 3), f32),
        ],
    )(g, hsa, ligand_pos_pert, nbr, p['We2'], p['be2'][None, :],
      p['Wx1'], p['bx1'][None, :], wx2row, bx2, vrow, cknn,
      aggb, xaggb, hlig,
      p['Wh1'][0:NODE_DIM], p['Wh1'][NODE_DIM:], p['bh1'][None, :],
      p['Wh2'], p['bh2'][None, :], p['Wd1'], p['bd1'][None, :],
      p['Wd2'], p['bd2'][None, :])

    return pn, pp, pe


# speedup vs baseline: 1.5283x; 1.0662x over previous
import jax
import jax.numpy as jnp
from jax import lax
from jax.experimental import pallas as pl
from jax.experimental.pallas import tpu as pltpu
from jax.experimental.pallas import tpu_sc as plsc

N_PROT = 4096
N_LIG = 1024
N_ALL = N_PROT + N_LIG
E_LIG = 2048
N_GRAPHS = 4
NODE_DIM = 128
EDGE_DIM = 64
TIME_DIM = 16
CLASS_DIM = 8
CLASS_EMB_DIM = 16
NUM_TIMESTEPS = 1000
K_NN = 32

LIG_BLK = 128
N_LIG_BLKS = N_LIG // LIG_BLK
EDGE_BLK = 512
N_EDGE_BLKS = E_LIG // EDGE_BLK
TBL_W = 256

SC_NC = 2
SC_NS = 16
SC_NW = SC_NC * SC_NS
GATHER_B = N_LIG * K_NN
GATHER_CH = 128
GATHER_NCH = GATHER_B // (SC_NW * GATHER_CH)


def _silu(x):
    return x * jax.nn.sigmoid(x)


def _embed_body(prot_ref, wpc_ref, lig_ref, wl_ref, lb_ref, g4_ref, e127_ref,
                a_ref, b_ref, be1_ref, lep_ref, wle_ref, eb_ref, ge4_ref,
                pos16_ref, hd_ref, hlig_ref, hsa_ref, eproj_ref):
    oh_lb = (lax.broadcasted_iota(jnp.int32, (N_LIG, N_GRAPHS), 1)
             == lb_ref[...]).astype(jnp.float32)
    hl = (jnp.dot(lig_ref[...], wl_ref[...], preferred_element_type=jnp.float32)
          + jnp.dot(oh_lb, g4_ref[...], preferred_element_type=jnp.float32)
          + e127_ref[...])
    hlig_ref[...] = hl
    hsa_ref[...] = jnp.dot(hl, a_ref[...],
                           preferred_element_type=jnp.float32) + be1_ref[...]
    hd_ref[0:N_PROT, 0:NODE_DIM] = jnp.dot(
        prot_ref[...], wpc_ref[...], preferred_element_type=jnp.float32)
    hd_ref[N_PROT:N_ALL, 0:NODE_DIM] = jnp.dot(
        hl, b_ref[...], preferred_element_type=jnp.float32)
    hd_ref[:, NODE_DIM:TBL_W] = pos16_ref[...]
    oh_eb = (lax.broadcasted_iota(jnp.int32, (E_LIG, N_GRAPHS), 1)
             == eb_ref[...]).astype(jnp.float32)
    eproj_ref[...] = (
        jnp.dot(lep_ref[...], wle_ref[...], preferred_element_type=jnp.float32)
        + jnp.dot(oh_eb, ge4_ref[...], preferred_element_type=jnp.float32))


def _knn_body(ligpos_ref, lb_ref, posT_ref, ab_ref, nbr_ref):
    xl = ligpos_ref[...]
    posT = posT_ref[...]
    sq_all = jnp.sum(posT * posT, axis=0, keepdims=True)
    sq_lig = jnp.sum(xl * xl, axis=1, keepdims=True)
    d2 = sq_lig + sq_all - 2.0 * jnp.dot(
        xl, posT, preferred_element_type=jnp.float32)
    col = lax.broadcasted_iota(jnp.int32, (N_LIG, N_ALL), 1)
    same_batch = ab_ref[...] == lb_ref[...]
    row_gid = N_PROT + lax.broadcasted_iota(
        jnp.int32, (N_LIG, N_ALL), 0)
    inf = jnp.float32(jnp.inf)
    d2 = jnp.where(jnp.logical_and(same_batch, col != row_gid), d2, inf)
    kcol = lax.broadcasted_iota(jnp.int32, (N_LIG, K_NN), 1)

    def step(k, carry):
        d2c, acc = carry
        mn = jnp.min(d2c, axis=1, keepdims=True)
        idx = jnp.min(jnp.where(d2c == mn, col, N_ALL), axis=1, keepdims=True)
        acc = jnp.where(kcol == k, idx, acc)
        return jnp.where(col == idx, inf, d2c), acc

    _, nbr = lax.fori_loop(
        0, K_NN, step,
        (d2, jnp.zeros((N_LIG, K_NN), jnp.int32)), unroll=8)
    nbr_ref[...] = nbr


def _sc_gather_body(tbl_ref, idx_ref, out_ref, idx_v, rows_v, sem):
    wid = lax.axis_index("s") * SC_NC + lax.axis_index("c")
    base_ch = wid * GATHER_NCH
    pltpu.sync_copy(idx_ref.at[pl.ds(base_ch, GATHER_NCH)], idx_v)
    for c in range(GATHER_NCH):
        pltpu.async_copy(tbl_ref.at[idx_v.at[c]], rows_v, sem).wait()
        pltpu.sync_copy(
            rows_v, out_ref.at[pl.ds((base_ch + c) * GATHER_CH, GATHER_CH)])


def _sc_gather(tbl, idx2):
    run = pl.kernel(
        _sc_gather_body,
        mesh=plsc.VectorSubcoreMesh(core_axis_name="c", subcore_axis_name="s"),
        out_type=jax.ShapeDtypeStruct((GATHER_B, TBL_W), jnp.float32),
        scratch_types=[
            pltpu.VMEM((GATHER_NCH, GATHER_CH), jnp.int32),
            pltpu.VMEM((GATHER_CH, TBL_W), jnp.float32),
            pltpu.SemaphoreType.DMA,
        ],
    )
    return run(tbl, idx2)


def _knn_msg_body(g_ref, hsa_ref, xl_ref, nbr_ref, we2_ref, be2_ref,
                  wx1_ref, bx1_ref, wx2_ref, bx2_ref, v_ref, ck_ref,
                  aggb_ref, xaggb_ref, hlig_ref, wh1a_ref, wh1b_ref, bh1_ref,
                  wh2_ref, bh2_ref, wd1_ref, bd1_ref, wd2_ref, bd2_ref,
                  pn_ref, pp_ref):
    g = g_ref[...]
    d = g[:, 0:NODE_DIM].reshape(LIG_BLK, K_NN, NODE_DIM)
    pnb = g[:, NODE_DIM:NODE_DIM + 3].reshape(LIG_BLK, K_NN, 3)
    xl = xl_ref[...]
    rel = xl[:, None, :] - pnb
    d2 = jnp.sum(rel * rel, axis=2, keepdims=True)
    m1 = _silu(hsa_ref[...][:, None, :] + d
               + d2 * v_ref[...][None] + ck_ref[...][None])
    m1f = m1.reshape(LIG_BLK * K_NN, NODE_DIM)
    m = _silu(jnp.dot(m1f, we2_ref[...],
                      preferred_element_type=jnp.float32) + be2_ref[...])
    u = _silu(jnp.dot(m, wx1_ref[...],
                      preferred_element_type=jnp.float32) + bx1_ref[...])
    w = jnp.tanh(jnp.sum(u * wx2_ref[...], axis=1, keepdims=True)
                 + bx2_ref[...])
    mask = (nbr_ref[...] < N_PROT).astype(jnp.float32)
    m3 = m.reshape(LIG_BLK, K_NN, NODE_DIM) * mask[:, :, None]
    agg = jnp.sum(m3, axis=1) + aggb_ref[...]
    wm = (w.reshape(LIG_BLK, K_NN, 1) * mask[:, :, None])
    xagg = jnp.sum(rel * wm, axis=1) + xaggb_ref[...]
    hl = hlig_ref[...]
    t1 = _silu(jnp.dot(hl, wh1a_ref[...], preferred_element_type=jnp.float32)
               + jnp.dot(agg, wh1b_ref[...],
                         preferred_element_type=jnp.float32) + bh1_ref[...])
    hn = hl + jnp.dot(t1, wh2_ref[...],
                      preferred_element_type=jnp.float32) + bh2_ref[...]
    pn1 = jax.nn.relu(jnp.dot(hn, wd1_ref[...],
                              preferred_element_type=jnp.float32) + bd1_ref[...])
    pn_ref[...] = jnp.dot(pn1, wd2_ref[...],
                          preferred_element_type=jnp.float32) + bd2_ref[...]
    pp_ref[...] = xl + xagg / K_NN


def _bond_body(srcc_ref, srcr_ref, dstc_ref, dstr_ref, hsa_ref, hdl_ref,
                    xlig_ref, eproj_ref, we2_ref, be2_ref, wx1_ref, bx1_ref,
                    wx2_ref, bx2_ref, v_ref, wed_ref, bed_ref, wq1_ref,
                    bq1_ref, wq2_ref, bq2_ref, pe_ref, aggb_ref, xaggb_ref):
    i = pl.program_id(0)
    oh_s = (lax.broadcasted_iota(jnp.int32, (EDGE_BLK, N_LIG), 1)
            == srcc_ref[...]).astype(jnp.float32)
    oh_d = (lax.broadcasted_iota(jnp.int32, (EDGE_BLK, N_LIG), 1)
            == dstc_ref[...]).astype(jnp.float32)
    hs = jnp.dot(oh_s, hsa_ref[...], preferred_element_type=jnp.float32)
    hd = jnp.dot(oh_d, hdl_ref[...], preferred_element_type=jnp.float32)
    xs = jnp.dot(oh_s, xlig_ref[...], preferred_element_type=jnp.float32)
    xd = jnp.dot(oh_d, xlig_ref[...], preferred_element_type=jnp.float32)
    rel = xs - xd
    d2 = jnp.sum(rel * rel, axis=1, keepdims=True)
    m1 = _silu(hs + hd + d2 * v_ref[...] + eproj_ref[...])
    m = _silu(jnp.dot(m1, we2_ref[...],
                      preferred_element_type=jnp.float32) + be2_ref[...])
    u = _silu(jnp.dot(m, wx1_ref[...],
                      preferred_element_type=jnp.float32) + bx1_ref[...])
    w = jnp.tanh(jnp.sum(u * wx2_ref[...], axis=1, keepdims=True)
                 + bx2_ref[...])
    oh_sT = (lax.broadcasted_iota(jnp.int32, (N_LIG, EDGE_BLK), 0)
             == srcr_ref[...]).astype(jnp.float32)

    @pl.when(i == 0)
    def _():
        aggb_ref[...] = jnp.zeros_like(aggb_ref)
        xaggb_ref[...] = jnp.zeros_like(xaggb_ref)

    aggb_ref[...] += jnp.dot(oh_sT, m, preferred_element_type=jnp.float32)
    xaggb_ref[...] += jnp.dot(oh_sT, rel * w,
                              preferred_element_type=jnp.float32)
    en = _silu(jnp.dot(m, wed_ref[...],
                       preferred_element_type=jnp.float32) + bed_ref[...])
    pe1 = jax.nn.relu(jnp.dot(en, wq1_ref[...],
                              preferred_element_type=jnp.float32) + bq1_ref[...])
    pe_ref[...] = jnp.dot(pe1, wq2_ref[...],
                          preferred_element_type=jnp.float32) + bq2_ref[...]


def kernel(protein_node, protein_pos, protein_batch, ligand_node_pert,
           ligand_pos_pert, ligand_batch, ligand_edge_pert, ligand_edge_index,
           ligand_edge_batch, t, lab, params):
    p = params
    f32 = jnp.float32

    offset = jnp.linspace(0.0, float(NUM_TIMESTEPS), TIME_DIM)
    coeff = -0.5 / (offset[1] - offset[0]) ** 2
    te4 = jnp.exp(coeff * (t[:, None].astype(f32) - offset[None, :]) ** 2)
    ch = lab @ p['Wc1'] + p['bc1']
    mu = ch.mean(-1, keepdims=True)
    var = ch.var(-1, keepdims=True)
    ch = (ch - mu) / jnp.sqrt(var + 1e-5) * p['ln_g'] + p['ln_b']
    ce4 = jax.nn.gelu(ch) @ p['Wc2'] + p['bc2']

    we1 = p['We1']
    A = we1[0:NODE_DIM]
    B = we1[NODE_DIM:2 * NODE_DIM]
    vrow = we1[2 * NODE_DIM:2 * NODE_DIM + 1]
    R = we1[2 * NODE_DIM + 1:]

    nd_l = NODE_DIM - 1 - TIME_DIM - CLASS_EMB_DIM
    g4 = jnp.concatenate(
        [jnp.zeros((N_GRAPHS, nd_l), f32), te4, ce4,
         jnp.zeros((N_GRAPHS, 1), f32)], axis=1)
    wl_pad = jnp.concatenate(
        [p['W_lig_node'], jnp.zeros((p['W_lig_node'].shape[0],
                                     NODE_DIM - nd_l), f32)], axis=1)
    e127 = jnp.zeros((1, NODE_DIM), f32).at[0, NODE_DIM - 1].set(1.0)
    wprot_pad = jnp.concatenate(
        [p['W_prot_node'], jnp.zeros((p['W_prot_node'].shape[0], 1), f32)],
        axis=1)
    wprot_comb = wprot_pad @ B
    ed_l = EDGE_DIM - TIME_DIM - CLASS_EMB_DIM
    wle_comb = p['W_lig_edge'] @ R[0:ed_l]
    ge4 = jnp.concatenate([te4, ce4], axis=1) @ R[ed_l:]
    cknn = (p['W_prot_edge'][0] @ R)[None, :]
    be1 = p['be1'][None, :]

    all_pos = jnp.concatenate([protein_pos, ligand_pos_pert], 0)
    all_batch = jnp.concatenate([protein_batch, ligand_batch], 0)
    lb_col = ligand_batch[:, None].astype(jnp.int32)
    eb_col = ligand_edge_batch[:, None].astype(jnp.int32)

    pos16 = jnp.concatenate(
        [all_pos, jnp.zeros((N_ALL, TBL_W - NODE_DIM - 3), f32)], axis=1)
    tbl, hlig, hsa, eproj = pl.pallas_call(
        _embed_body,
        out_shape=[
            jax.ShapeDtypeStruct((N_ALL, TBL_W), f32),
            jax.ShapeDtypeStruct((N_LIG, NODE_DIM), f32),
            jax.ShapeDtypeStruct((N_LIG, NODE_DIM), f32),
            jax.ShapeDtypeStruct((E_LIG, NODE_DIM), f32),
        ],
    )(protein_node, wprot_comb, ligand_node_pert, wl_pad, lb_col, g4, e127,
      A, B, be1, ligand_edge_pert, wle_comb, eb_col, ge4, pos16)

    nbr = pl.pallas_call(
        _knn_body,
        out_shape=jax.ShapeDtypeStruct((N_LIG, K_NN), jnp.int32),
    )(ligand_pos_pert, lb_col, all_pos.T, all_batch[None, :].astype(jnp.int32))

    idx2 = nbr.reshape(GATHER_B // GATHER_CH, GATHER_CH)
    g = _sc_gather(tbl, idx2)

    wx2row = p['Wx2'][:, 0][None, :]
    bx2 = p['bx2'][None, :]

    src_l = ligand_edge_index[0].astype(jnp.int32)
    dst_l = ligand_edge_index[1].astype(jnp.int32)
    hdl = tbl[N_PROT:, 0:NODE_DIM]
    wspec = pl.BlockSpec((NODE_DIM, NODE_DIM), lambda i: (0, 0))
    rspec = pl.BlockSpec((1, NODE_DIM), lambda i: (0, 0))
    pe, aggb, xaggb = pl.pallas_call(
        _bond_body,
        grid=(N_EDGE_BLKS,),
        in_specs=[
            pl.BlockSpec((EDGE_BLK, 1), lambda i: (i, 0)),
            pl.BlockSpec((1, EDGE_BLK), lambda i: (0, i)),
            pl.BlockSpec((EDGE_BLK, 1), lambda i: (i, 0)),
            pl.BlockSpec((1, EDGE_BLK), lambda i: (0, i)),
            pl.BlockSpec((N_LIG, NODE_DIM), lambda i: (0, 0)),
            pl.BlockSpec((N_LIG, NODE_DIM), lambda i: (0, 0)),
            pl.BlockSpec((N_LIG, 3), lambda i: (0, 0)),
            pl.BlockSpec((EDGE_BLK, NODE_DIM), lambda i: (i, 0)),
            wspec, rspec, wspec, rspec,
            rspec, pl.BlockSpec((1, 1), lambda i: (0, 0)), rspec,
            pl.BlockSpec((NODE_DIM, EDGE_DIM), lambda i: (0, 0)),
            pl.BlockSpec((1, EDGE_DIM), lambda i: (0, 0)),
            pl.BlockSpec((EDGE_DIM, EDGE_DIM), lambda i: (0, 0)),
            pl.BlockSpec((1, EDGE_DIM), lambda i: (0, 0)),
            pl.BlockSpec((EDGE_DIM, 5), lambda i: (0, 0)),
            pl.BlockSpec((1, 5), lambda i: (0, 0)),
        ],
        out_specs=[
            pl.BlockSpec((EDGE_BLK, 5), lambda i: (i, 0)),
            pl.BlockSpec((N_LIG, NODE_DIM), lambda i: (0, 0)),
            pl.BlockSpec((N_LIG, 3), lambda i: (0, 0)),
        ],
        out_shape=[
            jax.ShapeDtypeStruct((E_LIG, 5), f32),
            jax.ShapeDtypeStruct((N_LIG, NODE_DIM), f32),
            jax.ShapeDtypeStruct((N_LIG, 3), f32),
        ],
    )(src_l[:, None], src_l[None, :], dst_l[:, None], dst_l[None, :],
      hsa, hdl, ligand_pos_pert, eproj,
      p['We2'], p['be2'][None, :], p['Wx1'], p['bx1'][None, :],
      wx2row, bx2, vrow,
      p['Wed'], p['bed'][None, :], p['Wq1'], p['bq1'][None, :],
      p['Wq2'], p['bq2'][None, :])

    wspec2 = pl.BlockSpec((NODE_DIM, NODE_DIM), lambda i: (0, 0))
    rspec2 = pl.BlockSpec((1, NODE_DIM), lambda i: (0, 0))
    pn, pp = pl.pallas_call(
        _knn_msg_body,
        grid=(N_LIG_BLKS,),
        in_specs=[
            pl.BlockSpec((LIG_BLK * K_NN, TBL_W), lambda i: (i, 0)),
            pl.BlockSpec((LIG_BLK, NODE_DIM), lambda i: (i, 0)),
            pl.BlockSpec((LIG_BLK, 3), lambda i: (i, 0)),
            pl.BlockSpec((LIG_BLK, K_NN), lambda i: (i, 0)),
            wspec2, rspec2, wspec2, rspec2, rspec2,
            pl.BlockSpec((1, 1), lambda i: (0, 0)),
            rspec2, rspec2,
            pl.BlockSpec((LIG_BLK, NODE_DIM), lambda i: (i, 0)),
            pl.BlockSpec((LIG_BLK, 3), lambda i: (i, 0)),
            pl.BlockSpec((LIG_BLK, NODE_DIM), lambda i: (i, 0)),
            wspec2, wspec2, rspec2, wspec2, rspec2,
            wspec2, rspec2,
            pl.BlockSpec((NODE_DIM, p['Wd2'].shape[1]), lambda i: (0, 0)),
            pl.BlockSpec((1, p['Wd2'].shape[1]), lambda i: (0, 0)),
        ],
        out_specs=[
            pl.BlockSpec((LIG_BLK, p['Wd2'].shape[1]), lambda i: (i, 0)),
            pl.BlockSpec((LIG_BLK, 3), lambda i: (i, 0)),
        ],
        out_shape=[
            jax.ShapeDtypeStruct((N_LIG, p['Wd2'].shape[1]), f32),
            jax.ShapeDtypeStruct((N_LIG, 3), f32),
        ],
    )(g, hsa, ligand_pos_pert, nbr, p['We2'], p['be2'][None, :],
      p['Wx1'], p['bx1'][None, :], wx2row, bx2, vrow, cknn,
      aggb, xaggb, hlig,
      p['Wh1'][0:NODE_DIM], p['Wh1'][NODE_DIM:], p['bh1'][None, :],
      p['Wh2'], p['bh2'][None, :], p['Wd1'], p['bd1'][None, :],
      p['Wd2'], p['bd2'][None, :])

    return pn, pp, pe
